# Initial kernel scaffold; baseline (speedup 1.0000x reference)
import jax
import jax.numpy as jnp
from jax.experimental import pallas as pl


def kernel(xa, edge_index, starts, first_message, enc_W, enc_b, q_W, q_b, k_W, k_b, ro_W, ro_b, nm_W, nm_b, dec_W, dec_b):
    raise NotImplementedError("write your pallas kernel here")


# speedup vs baseline: 44.1749x; 44.1749x over previous
import jax
import jax.numpy as jnp
from jax import lax
from jax.experimental import pallas as pl
from jax.experimental.pallas import tpu as pltpu

N = 128
HID = 128
MSG = 64
HL = 10
HP = 16
MAXM = 1280
E = 512
OUTF = 16

_i32 = jnp.int32
_f32 = jnp.float32
_NEG = -1e30


def _lane(row, idx):
    ln = lax.broadcasted_iota(_i32, row.shape, 1)
    return jnp.sum(jnp.where(ln == idx, row, jnp.zeros_like(row)))


def _gwac_kernel(xa_ref, e0_ref, e1_ref, st_ref, fm_ref,
                 encW_ref, encb_ref, qW_ref, qb_ref, kW_ref, kb_ref,
                 roW_ref, rob_ref, nmW_ref, nmb_ref, decW_ref, decb_ref,
                 out_ref,
                 hist_ref, nmsg_ref, nbr_ref, roeff_ref,
                 cnt_ref, deg_ref, sl_ref, pdeg_ref, pnode_ref):
    f32 = _f32

    lane_e = lax.broadcasted_iota(_i32, (E, N), 1)
    oh0 = (e0_ref[:, :] == lane_e).astype(f32)
    oh1 = (e1_ref[:, :] == lane_e).astype(f32)
    c01 = lax.dot_general(oh0, oh1, (((0,), (0,)), ((), ())),
                          preferred_element_type=f32)
    adjf = ((c01 + c01.T) > 0).astype(f32)

    deg_ref[:, :] = jnp.sum(adjf, axis=0, keepdims=True).astype(_i32)

    ii = lax.broadcasted_iota(_i32, (N, N), 0)
    jj = lax.broadcasted_iota(_i32, (N, N), 1)
    tri = (ii <= jj).astype(f32)
    offs = (lax.dot_general(adjf, tri, (((1,), (0,)), ((), ())),
                            preferred_element_type=f32) - 1.0).astype(_i32)

    nbr_ref[:, 0, :] = jnp.where(adjf > 0, offs, -1)

    sm_row = (st_ref[:, :] != 0)
    smf = sm_row.astype(f32)
    spos = (lax.dot_general(smf, tri, (((1,), (0,)), ((), ())),
                            preferred_element_type=f32) - 1.0).astype(_i32)
    eq2 = jnp.logical_and(spos.T == jj, sm_row.T)
    sl_ref[:, :] = jnp.sum(jnp.where(eq2, ii, 0), axis=0, keepdims=True)
    S = jnp.sum(smf.astype(_i32))

    roeff_ref[:, :] = (roW_ref[0:HID, :] + roW_ref[HID:2 * HID, :] +
                       roW_ref[2 * HID:3 * HID, :] + roW_ref[3 * HID:4 * HID, :])

    encoded = jnp.dot(xa_ref[:, :], encW_ref[:, :],
                      preferred_element_type=f32) + encb_ref[:, :]
    hist_ref[:, :, :] = jnp.zeros((N, HP, HID), dtype=f32)
    hist_ref[:, 0, :] = encoded
    cnt_ref[:, :] = jnp.ones((1, N), dtype=_i32)

    lane_n = lax.broadcasted_iota(_i32, (1, N), 1)
    ri = lax.broadcasted_iota(_i32, (HP, 1), 0)
    inv_sqrt = jnp.float32(1.0 / (HID ** 0.5))

    def cond_fn(carry):
        h, tail, s, rem = carry
        return jnp.logical_and(h < tail, h < MAXM)

    def body_fn(carry):
        h, tail, s, rem = carry
        is_start = h < S

        def adv_cond(c):
            s_, r_ = c
            return jnp.logical_and(jnp.logical_not(is_start), r_ == 0)

        def adv_body(c):
            s_, _ = c
            s2 = s_ + 1
            return s2, pdeg_ref[s2]

        s, rem = lax.while_loop(adv_cond, adv_body, (s, rem))
        s_safe = jnp.maximum(s, 0)

        ps = pnode_ref[s_safe]
        j = pdeg_ref[s_safe] - rem
        crow = nbr_ref[ps]
        pushed_node = jnp.sum(jnp.where(crow == j, lane_n,
                                        jnp.zeros_like(lane_n)))
        sn = _lane(sl_ref[:, :], h)
        node = jnp.where(is_start, sn, pushed_node)
        fm_row = fm_ref[jnp.where(is_start, node, 0)]
        pm_row = nmsg_ref[s_safe]
        message = jnp.where(is_start, fm_row, pm_row)

        cnt = _lane(cnt_ref[:, :], node)
        feats = hist_ref[node]
        q = jnp.dot(message, qW_ref[:, :],
                    preferred_element_type=f32) + qb_ref[:, :]
        keys = jnp.dot(feats, kW_ref[:, :],
                       preferred_element_type=f32) + kb_ref[:, :]
        scores = lax.dot_general(keys, q, (((1,), (1,)), ((), ())),
                                 preferred_element_type=f32) * inv_sqrt
        vmask = ri < jnp.minimum(cnt, HL)
        sc = jnp.where(vmask, scores, _NEG)
        m = jnp.max(sc)
        p = jnp.where(vmask, jnp.exp(sc - m), 0.0)
        logits = p / jnp.sum(p)
        values = jnp.sum(logits * feats, axis=0, keepdims=True)
        newstate = jnp.dot(values, roeff_ref[:, :],
                           preferred_element_type=f32) + rob_ref[:, :]
        newmessage = (jnp.dot(newstate, nmW_ref[0:HID, :],
                              preferred_element_type=f32) +
                      jnp.dot(message, nmW_ref[HID:HID + MSG, :],
                              preferred_element_type=f32) + nmb_ref[:, :])

        nmsg_ref[h] = newmessage
        wi = lax.rem(cnt, HL)
        hist_ref[node] = jnp.where(ri == wi, newstate, feats)
        cnt_ref[:, :] = cnt_ref[:, :] + (lane_n == node).astype(_i32)
        dn = _lane(deg_ref[:, :], node)
        pdeg_ref[h] = dn
        pnode_ref[h] = node
        rem2 = jnp.where(is_start, rem, rem - 1)
        return h + 1, tail + dn, s, rem2

    lax.while_loop(cond_fn, body_fn,
                   (jnp.int32(0), S, jnp.int32(-1), jnp.int32(0)))

    cr = cnt_ref[:, :]
    li = lax.rem(cr - 1, HL).T
    ff2 = jnp.zeros((N, HID), dtype=f32)
    for r in range(HL):
        ff2 = ff2 + jnp.where(li == r, hist_ref[:, r, :], 0.0)
    ffr = jnp.sum(ff2, axis=0, keepdims=True)
    lg = jnp.dot(ffr, decW_ref[:, :],
                 preferred_element_type=f32) + decb_ref[:, :]
    mx = jnp.max(lg)
    out_ref[:, :] = lg - mx - jnp.log(jnp.sum(jnp.exp(lg - mx)))


def _run(xa, e0, e1, st, fm, enc_W, enc_b, q_W, q_b, k_W, k_b,
         ro_W, ro_b, nm_W, nm_b, dec_W, dec_b, *, interpret=False):
    return pl.pallas_call(
        _gwac_kernel,
        out_shape=jax.ShapeDtypeStruct((1, OUTF), _f32),
        scratch_shapes=[
            pltpu.VMEM((N, HP, HID), _f32),
            pltpu.VMEM((MAXM, 1, MSG), _f32),
            pltpu.VMEM((N, 1, N), _i32),
            pltpu.VMEM((HID, HID), _f32),
            pltpu.VMEM((1, N), _i32),
            pltpu.VMEM((1, N), _i32),
            pltpu.VMEM((1, N), _i32),
            pltpu.SMEM((MAXM,), _i32),
            pltpu.SMEM((MAXM,), _i32),
        ],
        interpret=interpret,
    )(xa, e0, e1, st, fm, enc_W, enc_b, q_W, q_b, k_W, k_b,
      ro_W, ro_b, nm_W, nm_b, dec_W, dec_b)


def kernel(xa, edge_index, starts, first_message, enc_W, enc_b, q_W, q_b,
           k_W, k_b, ro_W, ro_b, nm_W, nm_b, dec_W, dec_b):
    e0 = edge_index[0].reshape(E, 1)
    e1 = edge_index[1].reshape(E, 1)
    st = starts.reshape(1, N).astype(_i32)
    fm = first_message.reshape(N, 1, MSG)
    return _run(xa, e0, e1, st, fm,
                enc_W, enc_b.reshape(1, HID), q_W, q_b.reshape(1, HID),
                k_W, k_b.reshape(1, HID), ro_W, ro_b.reshape(1, HID),
                nm_W, nm_b.reshape(1, MSG), dec_W, dec_b.reshape(1, OUTF))


# speedup vs baseline: 50.7387x; 1.1486x over previous
import jax
import jax.numpy as jnp
from jax import lax
from jax.experimental import pallas as pl
from jax.experimental.pallas import tpu as pltpu

N = 128
HID = 128
MSG = 64
HL = 10
HP = 16
MAXM = 1280
E = 512
OUTF = 16

_i32 = jnp.int32
_f32 = jnp.float32
_NEG = -1e30


def _lane(row, idx):
    ln = lax.broadcasted_iota(_i32, row.shape, 1)
    return jnp.sum(jnp.where(ln == idx, row, jnp.zeros_like(row)))


def _gwac_kernel(xa_ref, e0_ref, e1_ref, st_ref, fm_ref,
                 encW_ref, encb_ref, qW_ref, qb_ref, kW_ref, kb_ref,
                 roW_ref, rob_ref, nmW_ref, nmb_ref, decW_ref, decb_ref,
                 out_ref,
                 hist_ref, nmsg_ref, nbr_ref, roeff_ref, aq_ref, cq_ref,
                 cnt_ref, deg_ref, sl_ref, pdeg_ref, pnode_ref,
                 cnts_ref, degs_ref, sls_ref):
    f32 = _f32

    lane_e = lax.broadcasted_iota(_i32, (E, N), 1)
    oh0 = (e0_ref[:, :] == lane_e).astype(f32)
    oh1 = (e1_ref[:, :] == lane_e).astype(f32)
    c01 = lax.dot_general(oh0, oh1, (((0,), (0,)), ((), ())),
                          preferred_element_type=f32)
    adjf = ((c01 + c01.T) > 0).astype(f32)

    deg_ref[:, :] = jnp.sum(adjf, axis=0, keepdims=True).astype(_i32)

    ii = lax.broadcasted_iota(_i32, (N, N), 0)
    jj = lax.broadcasted_iota(_i32, (N, N), 1)
    tri = (ii <= jj).astype(f32)
    offs = (lax.dot_general(adjf, tri, (((1,), (0,)), ((), ())),
                            preferred_element_type=f32) - 1.0).astype(_i32)

    nbr_ref[:, 0, :] = jnp.where(adjf > 0, offs, -1)

    sm_row = (st_ref[:, :] != 0)
    smf = sm_row.astype(f32)
    spos = (lax.dot_general(smf, tri, (((1,), (0,)), ((), ())),
                            preferred_element_type=f32) - 1.0).astype(_i32)
    eq2 = jnp.logical_and(spos.T == jj, sm_row.T)
    sl_ref[:, :] = jnp.sum(jnp.where(eq2, ii, 0), axis=0, keepdims=True)
    S = jnp.sum(smf.astype(_i32))

    roeff_ref[:, :] = (roW_ref[0:HID, :] + roW_ref[HID:2 * HID, :] +
                       roW_ref[2 * HID:3 * HID, :] + roW_ref[3 * HID:4 * HID, :])

    inv_sqrt = jnp.float32(1.0 / (HID ** 0.5))
    aq_ref[:, :] = lax.dot_general(kW_ref[:, :], qW_ref[:, :],
                                   (((1,), (1,)), ((), ())),
                                   preferred_element_type=f32) * inv_sqrt
    cq_ref[:, :] = lax.dot_general(qb_ref[:, :], kW_ref[:, :],
                                   (((1,), (1,)), ((), ())),
                                   preferred_element_type=f32) * inv_sqrt

    encoded = jnp.dot(xa_ref[:, :], encW_ref[:, :],
                      preferred_element_type=f32) + encb_ref[:, :]
    hist_ref[:, :, :] = jnp.zeros((N, HP, HID), dtype=f32)
    hist_ref[:, 0, :] = encoded
    cnt_ref[:, :] = jnp.ones((1, N), dtype=_i32)

    def _smem_init(n, _):
        cnts_ref[n] = 1
        degs_ref[n] = _lane(deg_ref[:, :], n)
        sls_ref[n] = _lane(sl_ref[:, :], n)
        return 0

    lax.fori_loop(0, N, _smem_init, 0)

    lane_n = lax.broadcasted_iota(_i32, (1, N), 1)
    ri = lax.broadcasted_iota(_i32, (HP, 1), 0)

    def cond_fn(carry):
        h, tail, s, rem = carry
        return jnp.logical_and(h < tail, h < MAXM)

    def body_fn(carry):
        h, tail, s, rem = carry
        is_start = h < S

        def adv_cond(c):
            s_, r_ = c
            return jnp.logical_and(jnp.logical_not(is_start), r_ == 0)

        def adv_body(c):
            s_, _ = c
            s2 = s_ + 1
            return s2, pdeg_ref[s2]

        s, rem = lax.while_loop(adv_cond, adv_body, (s, rem))
        s_safe = jnp.maximum(s, 0)

        ps = pnode_ref[s_safe]
        j = pdeg_ref[s_safe] - rem
        crow = nbr_ref[ps]
        pushed_node = jnp.sum(jnp.where(crow == j, lane_n,
                                        jnp.zeros_like(lane_n)))
        sn = sls_ref[jnp.minimum(h, N - 1)]
        node = jnp.where(is_start, sn, pushed_node)
        fm_row = fm_ref[jnp.where(is_start, node, 0)]
        pm_row = nmsg_ref[s_safe]
        message = jnp.where(is_start, fm_row, pm_row)

        cnt = cnts_ref[node]
        feats = hist_ref[node]
        kqT = lax.dot_general(message, aq_ref[:, :], (((1,), (1,)), ((), ())),
                              preferred_element_type=f32) + cq_ref[:, :]
        scores = lax.dot_general(feats, kqT, (((1,), (1,)), ((), ())),
                                 preferred_element_type=f32)
        vmask = ri < jnp.minimum(cnt, HL)
        sc = jnp.where(vmask, scores, _NEG)
        m = jnp.max(sc)
        p = jnp.where(vmask, jnp.exp(sc - m), 0.0)
        logits = p / jnp.sum(p)
        values = jnp.sum(logits * feats, axis=0, keepdims=True)
        newstate = jnp.dot(values, roeff_ref[:, :],
                           preferred_element_type=f32) + rob_ref[:, :]
        newmessage = (jnp.dot(newstate, nmW_ref[0:HID, :],
                              preferred_element_type=f32) +
                      jnp.dot(message, nmW_ref[HID:HID + MSG, :],
                              preferred_element_type=f32) + nmb_ref[:, :])

        nmsg_ref[h] = newmessage
        wi = lax.rem(cnt, HL)
        hist_ref[node] = jnp.where(ri == wi, newstate, feats)
        cnts_ref[node] = cnt + 1
        dn = degs_ref[node]
        pdeg_ref[h] = dn
        pnode_ref[h] = node
        rem2 = jnp.where(is_start, rem, rem - 1)
        return h + 1, tail + dn, s, rem2

    lax.while_loop(cond_fn, body_fn,
                   (jnp.int32(0), S, jnp.int32(-1), jnp.int32(0)))

    def _readout(n, acc):
        li = lax.rem(cnts_ref[n] - 1, HL)
        blk = hist_ref[n]
        return acc + jnp.sum(jnp.where(ri == li, blk, 0.0), axis=0,
                             keepdims=True)

    ffr = lax.fori_loop(0, N, _readout, jnp.zeros((1, HID), dtype=f32))
    lg = jnp.dot(ffr, decW_ref[:, :],
                 preferred_element_type=f32) + decb_ref[:, :]
    mx = jnp.max(lg)
    out_ref[:, :] = lg - mx - jnp.log(jnp.sum(jnp.exp(lg - mx)))


def _run(xa, e0, e1, st, fm, enc_W, enc_b, q_W, q_b, k_W, k_b,
         ro_W, ro_b, nm_W, nm_b, dec_W, dec_b, *, interpret=False):
    return pl.pallas_call(
        _gwac_kernel,
        out_shape=jax.ShapeDtypeStruct((1, OUTF), _f32),
        scratch_shapes=[
            pltpu.VMEM((N, HP, HID), _f32),
            pltpu.VMEM((MAXM, 1, MSG), _f32),
            pltpu.VMEM((N, 1, N), _i32),
            pltpu.VMEM((HID, HID), _f32),
            pltpu.VMEM((HID, MSG), _f32),
            pltpu.VMEM((1, HID), _f32),
            pltpu.VMEM((1, N), _i32),
            pltpu.VMEM((1, N), _i32),
            pltpu.VMEM((1, N), _i32),
            pltpu.SMEM((MAXM,), _i32),
            pltpu.SMEM((MAXM,), _i32),
            pltpu.SMEM((N,), _i32),
            pltpu.SMEM((N,), _i32),
            pltpu.SMEM((N,), _i32),
        ],
        interpret=interpret,
    )(xa, e0, e1, st, fm, enc_W, enc_b, q_W, q_b, k_W, k_b,
      ro_W, ro_b, nm_W, nm_b, dec_W, dec_b)


def kernel(xa, edge_index, starts, first_message, enc_W, enc_b, q_W, q_b,
           k_W, k_b, ro_W, ro_b, nm_W, nm_b, dec_W, dec_b):
    e0 = edge_index[0].reshape(E, 1)
    e1 = edge_index[1].reshape(E, 1)
    st = starts.reshape(1, N).astype(_i32)
    fm = first_message.reshape(N, 1, MSG)
    return _run(xa, e0, e1, st, fm,
                enc_W, enc_b.reshape(1, HID), q_W, q_b.reshape(1, HID),
                k_W, k_b.reshape(1, HID), ro_W, ro_b.reshape(1, HID),
                nm_W, nm_b.reshape(1, MSG), dec_W, dec_b.reshape(1, OUTF))


# speedup vs baseline: 89.9484x; 1.7728x over previous
import jax
import jax.numpy as jnp
from jax import lax
from jax.experimental import pallas as pl
from jax.experimental.pallas import tpu as pltpu

N = 128
HID = 128
MSG = 64
HL = 10
HP = 16
MAXM = 1280
E = 512
OUTF = 16
BW = 8

_i32 = jnp.int32
_f32 = jnp.float32
_NEG = -1e30


def _lane(row, idx):
    ln = lax.broadcasted_iota(_i32, row.shape, 1)
    return jnp.sum(jnp.where(ln == idx, row, jnp.zeros_like(row)))


def _gwac_kernel(xa_ref, e0_ref, e1_ref, st_ref, fm_ref,
                 encW_ref, encb_ref, qW_ref, qb_ref, kW_ref, kb_ref,
                 roW_ref, rob_ref, nmW_ref, nmb_ref, decW_ref, decb_ref,
                 out_ref,
                 hist_ref, nmsg_ref, nbr_ref, roeff_ref, aq_ref, cq_ref,
                 deg_ref, sl_ref, bf_ref, bm_ref,
                 cnts_ref, degs_ref, sls_ref, lastlvl_ref,
                 pnode_ref, msrc_ref, cntt_ref, lvl_ref,
                 lcnt_ref, wptr_ref, order_ref, bstart_ref, bcnt_ref):
    f32 = _f32

    lane_e = lax.broadcasted_iota(_i32, (E, N), 1)
    oh0 = (e0_ref[:, :] == lane_e).astype(f32)
    oh1 = (e1_ref[:, :] == lane_e).astype(f32)
    c01 = lax.dot_general(oh0, oh1, (((0,), (0,)), ((), ())),
                          preferred_element_type=f32)
    adjf = ((c01 + c01.T) > 0).astype(f32)

    deg_ref[:, :] = jnp.sum(adjf, axis=0, keepdims=True).astype(_i32)

    ii = lax.broadcasted_iota(_i32, (N, N), 0)
    jj = lax.broadcasted_iota(_i32, (N, N), 1)
    tri = (ii <= jj).astype(f32)
    offs = (lax.dot_general(adjf, tri, (((1,), (0,)), ((), ())),
                            preferred_element_type=f32) - 1.0).astype(_i32)

    nbr_ref[:, 0, :] = jnp.where(adjf > 0, offs, -1)

    sm_row = (st_ref[:, :] != 0)
    smf = sm_row.astype(f32)
    spos = (lax.dot_general(smf, tri, (((1,), (0,)), ((), ())),
                            preferred_element_type=f32) - 1.0).astype(_i32)
    eq2 = jnp.logical_and(spos.T == jj, sm_row.T)
    sl_ref[:, :] = jnp.sum(jnp.where(eq2, ii, 0), axis=0, keepdims=True)
    S = jnp.sum(smf.astype(_i32))

    roeff_ref[:, :] = (roW_ref[0:HID, :] + roW_ref[HID:2 * HID, :] +
                       roW_ref[2 * HID:3 * HID, :] + roW_ref[3 * HID:4 * HID, :])

    inv_sqrt = jnp.float32(1.0 / (HID ** 0.5))
    aq_ref[:, :] = lax.dot_general(kW_ref[:, :], qW_ref[:, :],
                                   (((1,), (1,)), ((), ())),
                                   preferred_element_type=f32) * inv_sqrt
    cq_ref[:, :] = lax.dot_general(qb_ref[:, :], kW_ref[:, :],
                                   (((1,), (1,)), ((), ())),
                                   preferred_element_type=f32) * inv_sqrt

    encoded = jnp.dot(xa_ref[:, :], encW_ref[:, :],
                      preferred_element_type=f32) + encb_ref[:, :]
    hist_ref[:, :, :] = jnp.zeros((N + 1, HP, HID), dtype=f32)
    hist_ref[0:N, 0, :] = encoded

    def _smem_init_n(n, _):
        cnts_ref[n] = 1
        lastlvl_ref[n] = -1
        degs_ref[n] = _lane(deg_ref[:, :], n)
        sls_ref[n] = _lane(sl_ref[:, :], n)
        return 0

    lax.fori_loop(0, N, _smem_init_n, 0)

    def _smem_init_m(i, _):
        lcnt_ref[i] = 0
        return 0

    lax.fori_loop(0, MAXM, _smem_init_m, 0)

    lane_n = lax.broadcasted_iota(_i32, (1, N), 1)
    ri = lax.broadcasted_iota(_i32, (HP, 1), 0)

    def a_cond(carry):
        h, tail, s, rem, maxlvl = carry
        return jnp.logical_and(h < tail, h < MAXM)

    def a_body(carry):
        h, tail, s, rem, maxlvl = carry
        is_start = h < S

        def adv_cond(c):
            s_, r_ = c
            return jnp.logical_and(jnp.logical_not(is_start), r_ == 0)

        def adv_body(c):
            s_, _ = c
            s2 = s_ + 1
            return s2, degs_ref[pnode_ref[s2]]

        s, rem = lax.while_loop(adv_cond, adv_body, (s, rem))
        s_safe = jnp.maximum(s, 0)

        ps = pnode_ref[s_safe]
        j = degs_ref[ps] - rem
        crow = nbr_ref[ps]
        pushed_node = jnp.sum(jnp.where(crow == j, lane_n,
                                        jnp.zeros_like(lane_n)))
        sn = sls_ref[jnp.minimum(h, N - 1)]
        node = jnp.where(is_start, sn, pushed_node)

        cnt = cnts_ref[node]
        cnts_ref[node] = cnt + 1
        cntt_ref[h] = cnt
        pnode_ref[h] = node
        msrc_ref[h] = jnp.where(is_start, -1, s_safe)

        lvl_src = jnp.where(is_start, -1, lvl_ref[s_safe])
        mylvl = jnp.maximum(lvl_src, lastlvl_ref[node]) + 1
        lvl_ref[h] = mylvl
        lastlvl_ref[node] = mylvl
        lcnt_ref[mylvl] = lcnt_ref[mylvl] + 1
        maxlvl = jnp.maximum(maxlvl, mylvl)

        dn = degs_ref[node]
        rem2 = jnp.where(is_start, rem, rem - 1)
        return h + 1, tail + dn, s, rem2, maxlvl

    T, _, _, _, maxlvl = lax.while_loop(
        a_cond, a_body,
        (jnp.int32(0), S, jnp.int32(-1), jnp.int32(0), jnp.int32(-1)))
    nlev = maxlvl + 1

    def _wp_body(l, pos):
        wptr_ref[l] = pos
        return pos + lcnt_ref[l]

    lax.fori_loop(0, nlev, _wp_body, jnp.int32(0))

    def _fill_body(t, _):
        L = lvl_ref[t]
        w = wptr_ref[L]
        order_ref[w] = t
        wptr_ref[L] = w + 1
        return 0

    lax.fori_loop(0, T, _fill_body, 0)

    def b_cond(carry):
        l, done, pos, nb = carry
        return l < nlev

    def b_body(carry):
        l, done, pos, nb = carry
        c = lcnt_ref[l]
        take = jnp.minimum(BW, c - done)
        bstart_ref[nb] = pos
        bcnt_ref[nb] = take
        done2 = done + take
        adv = done2 >= c
        return (jnp.where(adv, l + 1, l), jnp.where(adv, 0, done2),
                pos + take, nb + 1)

    _, _, _, NB = lax.while_loop(
        b_cond, b_body,
        (jnp.int32(0), jnp.int32(0), jnp.int32(0), jnp.int32(0)))

    exr = lax.broadcasted_iota(_i32, (HP * BW, BW), 0)
    exc = lax.broadcasted_iota(_i32, (HP * BW, BW), 1)
    EX = ((exr // HP) == exc).astype(f32)
    rmod = lax.rem(lax.broadcasted_iota(_i32, (HP * BW, 1), 0), HP)

    def p_body(b, _):
        p0 = bstart_ref[b]
        bc = bcnt_ref[b]
        nodes = []
        ts = []
        cnts = []
        for i in range(BW):
            valid = i < bc
            oi = order_ref[jnp.minimum(p0 + i, MAXM - 1)]
            t_i = jnp.where(valid, oi, 0)
            node = jnp.where(valid, pnode_ref[t_i], N)
            src = jnp.where(valid, msrc_ref[t_i], -1)
            cnt = jnp.where(valid, cntt_ref[t_i], 1)
            feats = hist_ref[node]
            bf_ref[i * HP:(i + 1) * HP, :] = feats
            fmr = fm_ref[jnp.minimum(node, N - 1)]
            pmr = nmsg_ref[jnp.maximum(src, 0)]
            bm_ref[i:i + 1, :] = jnp.where(src >= 0, pmr, fmr)
            nodes.append(node)
            ts.append(jnp.where(valid, t_i, MAXM))
            cnts.append(cnt)

        bf = bf_ref[:, :]
        bm = bm_ref[:, :]
        kqT = lax.dot_general(bm, aq_ref[:, :], (((1,), (1,)), ((), ())),
                              preferred_element_type=f32) + cq_ref[:, :]
        kqE = jnp.dot(EX, kqT, preferred_element_type=f32)
        st = jnp.sum(bf * kqE, axis=1, keepdims=True)

        cnt8 = jnp.concatenate(
            [jnp.minimum(c, HL).astype(f32).reshape(1, 1) for c in cnts],
            axis=0)
        cntE = jnp.dot(EX, cnt8, preferred_element_type=f32)
        vmask = rmod.astype(f32) < cntE
        sc = jnp.where(vmask, st, _NEG)
        m8 = jnp.concatenate(
            [jnp.max(sc[i * HP:(i + 1) * HP]).reshape(1, 1) for i in range(BW)],
            axis=0)
        mE = jnp.dot(EX, m8, preferred_element_type=f32)
        p = jnp.where(vmask, jnp.exp(sc - mE), 0.0)
        d8 = lax.dot_general(EX, p, (((0,), (0,)), ((), ())),
                             preferred_element_type=f32)
        dE = jnp.dot(EX, d8, preferred_element_type=f32)
        w = bf * (p / dE)
        V8 = lax.dot_general(EX, w, (((0,), (0,)), ((), ())),
                             preferred_element_type=f32)
        ns8 = jnp.dot(V8, roeff_ref[:, :],
                      preferred_element_type=f32) + rob_ref[:, :]
        nm8 = (jnp.dot(ns8, nmW_ref[0:HID, :], preferred_element_type=f32) +
               jnp.dot(bm, nmW_ref[HID:HID + MSG, :],
                       preferred_element_type=f32) + nmb_ref[:, :])

        for i in range(BW):
            wi = lax.rem(cnts[i], HL)
            blend = jnp.where(ri == wi, ns8[i:i + 1, :],
                              bf[i * HP:(i + 1) * HP, :])
            hist_ref[nodes[i]] = blend
            nmsg_ref[ts[i]] = nm8[i:i + 1, :]
        return 0

    lax.fori_loop(0, NB, p_body, 0)

    def _readout(n, acc):
        li = lax.rem(cnts_ref[n] - 1, HL)
        blk = hist_ref[n]
        return acc + jnp.sum(jnp.where(ri == li, blk, 0.0), axis=0,
                             keepdims=True)

    ffr = lax.fori_loop(0, N, _readout, jnp.zeros((1, HID), dtype=f32))
    lg = jnp.dot(ffr, decW_ref[:, :],
                 preferred_element_type=f32) + decb_ref[:, :]
    mx = jnp.max(lg)
    out_ref[:, :] = lg - mx - jnp.log(jnp.sum(jnp.exp(lg - mx)))


def _run(xa, e0, e1, st, fm, enc_W, enc_b, q_W, q_b, k_W, k_b,
         ro_W, ro_b, nm_W, nm_b, dec_W, dec_b, *, interpret=False):
    return pl.pallas_call(
        _gwac_kernel,
        out_shape=jax.ShapeDtypeStruct((1, OUTF), _f32),
        scratch_shapes=[
            pltpu.VMEM((N + 1, HP, HID), _f32),
            pltpu.VMEM((MAXM + 1, 1, MSG), _f32),
            pltpu.VMEM((N, 1, N), _i32),
            pltpu.VMEM((HID, HID), _f32),
            pltpu.VMEM((HID, MSG), _f32),
            pltpu.VMEM((1, HID), _f32),
            pltpu.VMEM((1, N), _i32),
            pltpu.VMEM((1, N), _i32),
            pltpu.VMEM((HP * BW, HID), _f32),
            pltpu.VMEM((BW, MSG), _f32),
            pltpu.SMEM((N,), _i32),
            pltpu.SMEM((N,), _i32),
            pltpu.SMEM((N,), _i32),
            pltpu.SMEM((N,), _i32),
            pltpu.SMEM((MAXM,), _i32),
            pltpu.SMEM((MAXM,), _i32),
            pltpu.SMEM((MAXM,), _i32),
            pltpu.SMEM((MAXM,), _i32),
            pltpu.SMEM((MAXM,), _i32),
            pltpu.SMEM((MAXM,), _i32),
            pltpu.SMEM((MAXM,), _i32),
            pltpu.SMEM((MAXM,), _i32),
            pltpu.SMEM((MAXM,), _i32),
        ],
        interpret=interpret,
    )(xa, e0, e1, st, fm, enc_W, enc_b, q_W, q_b, k_W, k_b,
      ro_W, ro_b, nm_W, nm_b, dec_W, dec_b)


def kernel(xa, edge_index, starts, first_message, enc_W, enc_b, q_W, q_b,
           k_W, k_b, ro_W, ro_b, nm_W, nm_b, dec_W, dec_b):
    e0 = edge_index[0].reshape(E, 1)
    e1 = edge_index[1].reshape(E, 1)
    st = starts.reshape(1, N).astype(_i32)
    fm = first_message.reshape(N, 1, MSG)
    return _run(xa, e0, e1, st, fm,
                enc_W, enc_b.reshape(1, HID), q_W, q_b.reshape(1, HID),
                k_W, k_b.reshape(1, HID), ro_W, ro_b.reshape(1, HID),
                nm_W, nm_b.reshape(1, MSG), dec_W, dec_b.reshape(1, OUTF))


# speedup vs baseline: 111.1509x; 1.2357x over previous
import jax
import jax.numpy as jnp
from jax import lax
from jax.experimental import pallas as pl
from jax.experimental.pallas import tpu as pltpu

N = 128
HID = 128
MSG = 64
HL = 10
HP = 16
MAXM = 1280
E = 512
OUTF = 16
BW = 16

_i32 = jnp.int32
_f32 = jnp.float32
_NEG = -1e30


def _lane(row, idx):
    ln = lax.broadcasted_iota(_i32, row.shape, 1)
    return jnp.sum(jnp.where(ln == idx, row, jnp.zeros_like(row)))


def _gwac_kernel(xa_ref, e0_ref, e1_ref, st_ref, fm_ref,
                 encW_ref, encb_ref, qW_ref, qb_ref, kW_ref, kb_ref,
                 roW_ref, rob_ref, nmW_ref, nmb_ref, decW_ref, decb_ref,
                 out_ref,
                 hist_ref, nmsg_ref, nbr_ref, roeff_ref, aq_ref, cq_ref,
                 deg_ref, sl_ref, bf_ref, bm_ref,
                 cnts_ref, degs_ref, sls_ref, lastlvl_ref,
                 pnode_ref, msrc_ref, cntt_ref, lvl_ref,
                 lcnt_ref, wptr_ref, order_ref, bstart_ref, bcnt_ref):
    f32 = _f32

    lane_e = lax.broadcasted_iota(_i32, (E, N), 1)
    oh0 = (e0_ref[:, :] == lane_e).astype(f32)
    oh1 = (e1_ref[:, :] == lane_e).astype(f32)
    c01 = lax.dot_general(oh0, oh1, (((0,), (0,)), ((), ())),
                          preferred_element_type=f32)
    adjf = ((c01 + c01.T) > 0).astype(f32)

    deg_ref[:, :] = jnp.sum(adjf, axis=0, keepdims=True).astype(_i32)

    ii = lax.broadcasted_iota(_i32, (N, N), 0)
    jj = lax.broadcasted_iota(_i32, (N, N), 1)
    tri = (ii <= jj).astype(f32)
    offs = (lax.dot_general(adjf, tri, (((1,), (0,)), ((), ())),
                            preferred_element_type=f32) - 1.0).astype(_i32)

    nbr_ref[:, 0, :] = jnp.where(adjf > 0, offs, -1)

    sm_row = (st_ref[:, :] != 0)
    smf = sm_row.astype(f32)
    spos = (lax.dot_general(smf, tri, (((1,), (0,)), ((), ())),
                            preferred_element_type=f32) - 1.0).astype(_i32)
    eq2 = jnp.logical_and(spos.T == jj, sm_row.T)
    sl_ref[:, :] = jnp.sum(jnp.where(eq2, ii, 0), axis=0, keepdims=True)
    S = jnp.sum(smf.astype(_i32))

    roeff_ref[:, :] = (roW_ref[0:HID, :] + roW_ref[HID:2 * HID, :] +
                       roW_ref[2 * HID:3 * HID, :] + roW_ref[3 * HID:4 * HID, :])

    inv_sqrt = jnp.float32(1.0 / (HID ** 0.5))
    aq_ref[:, :] = lax.dot_general(kW_ref[:, :], qW_ref[:, :],
                                   (((1,), (1,)), ((), ())),
                                   preferred_element_type=f32) * inv_sqrt
    cq_ref[:, :] = lax.dot_general(qb_ref[:, :], kW_ref[:, :],
                                   (((1,), (1,)), ((), ())),
                                   preferred_element_type=f32) * inv_sqrt

    encoded = jnp.dot(xa_ref[:, :], encW_ref[:, :],
                      preferred_element_type=f32) + encb_ref[:, :]
    hist_ref[:, :, :] = jnp.zeros((N + 1, HP, HID), dtype=f32)
    hist_ref[0:N, 0, :] = encoded

    def _smem_init_n(n, _):
        cnts_ref[n] = 1
        lastlvl_ref[n] = -1
        degs_ref[n] = _lane(deg_ref[:, :], n)
        sls_ref[n] = _lane(sl_ref[:, :], n)
        return 0

    lax.fori_loop(0, N, _smem_init_n, 0)

    def _smem_init_m(i, _):
        lcnt_ref[i] = 0
        return 0

    lax.fori_loop(0, MAXM, _smem_init_m, 0)

    lane_n = lax.broadcasted_iota(_i32, (1, N), 1)
    ri = lax.broadcasted_iota(_i32, (HP, 1), 0)

    def a_cond(carry):
        h, tail, s, rem, maxlvl = carry
        return jnp.logical_and(h < tail, h < MAXM)

    def a_body(carry):
        h, tail, s, rem, maxlvl = carry
        is_start = h < S

        def adv_cond(c):
            s_, r_ = c
            return jnp.logical_and(jnp.logical_not(is_start), r_ == 0)

        def adv_body(c):
            s_, _ = c
            s2 = s_ + 1
            return s2, degs_ref[pnode_ref[s2]]

        s, rem = lax.while_loop(adv_cond, adv_body, (s, rem))
        s_safe = jnp.maximum(s, 0)

        ps = pnode_ref[s_safe]
        j = degs_ref[ps] - rem
        crow = nbr_ref[ps]
        pushed_node = jnp.sum(jnp.where(crow == j, lane_n,
                                        jnp.zeros_like(lane_n)))
        sn = sls_ref[jnp.minimum(h, N - 1)]
        node = jnp.where(is_start, sn, pushed_node)

        cnt = cnts_ref[node]
        cnts_ref[node] = cnt + 1
        cntt_ref[h] = cnt
        pnode_ref[h] = node
        msrc_ref[h] = jnp.where(is_start, -1, s_safe)

        lvl_src = jnp.where(is_start, -1, lvl_ref[s_safe])
        mylvl = jnp.maximum(lvl_src, lastlvl_ref[node]) + 1
        lvl_ref[h] = mylvl
        lastlvl_ref[node] = mylvl
        lcnt_ref[mylvl] = lcnt_ref[mylvl] + 1
        maxlvl = jnp.maximum(maxlvl, mylvl)

        dn = degs_ref[node]
        rem2 = jnp.where(is_start, rem, rem - 1)
        return h + 1, tail + dn, s, rem2, maxlvl

    T, _, _, _, maxlvl = lax.while_loop(
        a_cond, a_body,
        (jnp.int32(0), S, jnp.int32(-1), jnp.int32(0), jnp.int32(-1)))
    nlev = maxlvl + 1

    def _wp_body(l, pos):
        wptr_ref[l] = pos
        return pos + lcnt_ref[l]

    lax.fori_loop(0, nlev, _wp_body, jnp.int32(0))

    def _fill_body(t, _):
        L = lvl_ref[t]
        w = wptr_ref[L]
        order_ref[w] = t
        wptr_ref[L] = w + 1
        return 0

    lax.fori_loop(0, T, _fill_body, 0)

    def b_cond(carry):
        l, done, pos, nb = carry
        return l < nlev

    def b_body(carry):
        l, done, pos, nb = carry
        c = lcnt_ref[l]
        take = jnp.minimum(BW, c - done)
        bstart_ref[nb] = pos
        bcnt_ref[nb] = take
        done2 = done + take
        adv = done2 >= c
        return (jnp.where(adv, l + 1, l), jnp.where(adv, 0, done2),
                pos + take, nb + 1)

    _, _, _, NB = lax.while_loop(
        b_cond, b_body,
        (jnp.int32(0), jnp.int32(0), jnp.int32(0), jnp.int32(0)))

    exr = lax.broadcasted_iota(_i32, (HP * BW, BW), 0)
    exc = lax.broadcasted_iota(_i32, (HP * BW, BW), 1)
    EX = ((exr // HP) == exc).astype(f32)
    rmod = lax.rem(lax.broadcasted_iota(_i32, (HP * BW, 1), 0), HP)

    def p_body(b, _):
        p0 = bstart_ref[b]
        bc = bcnt_ref[b]
        nodes = []
        ts = []
        cnts = []
        for i in range(BW):
            valid = i < bc
            oi = order_ref[jnp.minimum(p0 + i, MAXM - 1)]
            t_i = jnp.where(valid, oi, 0)
            node = jnp.where(valid, pnode_ref[t_i], N)
            src = jnp.where(valid, msrc_ref[t_i], -1)
            cnt = jnp.where(valid, cntt_ref[t_i], 1)
            feats = hist_ref[node]
            bf_ref[i * HP:(i + 1) * HP, :] = feats
            fmr = fm_ref[jnp.minimum(node, N - 1)]
            pmr = nmsg_ref[jnp.maximum(src, 0)]
            bm_ref[i:i + 1, :] = jnp.where(src >= 0, pmr, fmr)
            nodes.append(node)
            ts.append(jnp.where(valid, t_i, MAXM))
            cnts.append(cnt)

        bf = bf_ref[:, :]
        bm = bm_ref[:, :]
        kqT = lax.dot_general(bm, aq_ref[:, :], (((1,), (1,)), ((), ())),
                              preferred_element_type=f32) + cq_ref[:, :]
        kqE = jnp.dot(EX, kqT, preferred_element_type=f32)
        st = jnp.sum(bf * kqE, axis=1, keepdims=True)

        cnt8 = jnp.concatenate(
            [jnp.minimum(c, HL).astype(f32).reshape(1, 1) for c in cnts],
            axis=0)
        cntE = jnp.dot(EX, cnt8, preferred_element_type=f32)
        vmask = rmod.astype(f32) < cntE
        sc = jnp.where(vmask, st, _NEG)
        m8 = jnp.concatenate(
            [jnp.max(sc[i * HP:(i + 1) * HP]).reshape(1, 1) for i in range(BW)],
            axis=0)
        mE = jnp.dot(EX, m8, preferred_element_type=f32)
        p = jnp.where(vmask, jnp.exp(sc - mE), 0.0)
        d8 = lax.dot_general(EX, p, (((0,), (0,)), ((), ())),
                             preferred_element_type=f32)
        dE = jnp.dot(EX, d8, preferred_element_type=f32)
        w = bf * (p / dE)
        V8 = lax.dot_general(EX, w, (((0,), (0,)), ((), ())),
                             preferred_element_type=f32)
        ns8 = jnp.dot(V8, roeff_ref[:, :],
                      preferred_element_type=f32) + rob_ref[:, :]
        nm8 = (jnp.dot(ns8, nmW_ref[0:HID, :], preferred_element_type=f32) +
               jnp.dot(bm, nmW_ref[HID:HID + MSG, :],
                       preferred_element_type=f32) + nmb_ref[:, :])

        for i in range(BW):
            wi = lax.rem(cnts[i], HL)
            blend = jnp.where(ri == wi, ns8[i:i + 1, :],
                              bf[i * HP:(i + 1) * HP, :])
            hist_ref[nodes[i]] = blend
            nmsg_ref[ts[i]] = nm8[i:i + 1, :]
        return 0

    lax.fori_loop(0, NB, p_body, 0)

    def _readout(n, acc):
        li = lax.rem(cnts_ref[n] - 1, HL)
        blk = hist_ref[n]
        return acc + jnp.sum(jnp.where(ri == li, blk, 0.0), axis=0,
                             keepdims=True)

    ffr = lax.fori_loop(0, N, _readout, jnp.zeros((1, HID), dtype=f32))
    lg = jnp.dot(ffr, decW_ref[:, :],
                 preferred_element_type=f32) + decb_ref[:, :]
    mx = jnp.max(lg)
    out_ref[:, :] = lg - mx - jnp.log(jnp.sum(jnp.exp(lg - mx)))


def _run(xa, e0, e1, st, fm, enc_W, enc_b, q_W, q_b, k_W, k_b,
         ro_W, ro_b, nm_W, nm_b, dec_W, dec_b, *, interpret=False):
    return pl.pallas_call(
        _gwac_kernel,
        out_shape=jax.ShapeDtypeStruct((1, OUTF), _f32),
        scratch_shapes=[
            pltpu.VMEM((N + 1, HP, HID), _f32),
            pltpu.VMEM((MAXM + 1, 1, MSG), _f32),
            pltpu.VMEM((N, 1, N), _i32),
            pltpu.VMEM((HID, HID), _f32),
            pltpu.VMEM((HID, MSG), _f32),
            pltpu.VMEM((1, HID), _f32),
            pltpu.VMEM((1, N), _i32),
            pltpu.VMEM((1, N), _i32),
            pltpu.VMEM((HP * BW, HID), _f32),
            pltpu.VMEM((BW, MSG), _f32),
            pltpu.SMEM((N,), _i32),
            pltpu.SMEM((N,), _i32),
            pltpu.SMEM((N,), _i32),
            pltpu.SMEM((N,), _i32),
            pltpu.SMEM((MAXM,), _i32),
            pltpu.SMEM((MAXM,), _i32),
            pltpu.SMEM((MAXM,), _i32),
            pltpu.SMEM((MAXM,), _i32),
            pltpu.SMEM((MAXM,), _i32),
            pltpu.SMEM((MAXM,), _i32),
            pltpu.SMEM((MAXM,), _i32),
            pltpu.SMEM((MAXM,), _i32),
            pltpu.SMEM((MAXM,), _i32),
        ],
        interpret=interpret,
    )(xa, e0, e1, st, fm, enc_W, enc_b, q_W, q_b, k_W, k_b,
      ro_W, ro_b, nm_W, nm_b, dec_W, dec_b)


def kernel(xa, edge_index, starts, first_message, enc_W, enc_b, q_W, q_b,
           k_W, k_b, ro_W, ro_b, nm_W, nm_b, dec_W, dec_b):
    e0 = edge_index[0].reshape(E, 1)
    e1 = edge_index[1].reshape(E, 1)
    st = starts.reshape(1, N).astype(_i32)
    fm = first_message.reshape(N, 1, MSG)
    return _run(xa, e0, e1, st, fm,
                enc_W, enc_b.reshape(1, HID), q_W, q_b.reshape(1, HID),
                k_W, k_b.reshape(1, HID), ro_W, ro_b.reshape(1, HID),
                nm_W, nm_b.reshape(1, MSG), dec_W, dec_b.reshape(1, OUTF))


# speedup vs baseline: 115.4306x; 1.0385x over previous
import jax
import jax.numpy as jnp
from jax import lax
from jax.experimental import pallas as pl
from jax.experimental.pallas import tpu as pltpu

N = 128
HID = 128
MSG = 64
HL = 10
HP = 16
MAXM = 1280
E = 512
OUTF = 16
BW = 16

_i32 = jnp.int32
_f32 = jnp.float32
_NEG = -1e30


def _lane(row, idx):
    ln = lax.broadcasted_iota(_i32, row.shape, 1)
    return jnp.sum(jnp.where(ln == idx, row, jnp.zeros_like(row)))


def _gwac_kernel(xa_ref, e0_ref, e1_ref, st_ref, fm_ref,
                 encW_ref, encb_ref, qW_ref, qb_ref, kW_ref, kb_ref,
                 roW_ref, rob_ref, nmW_ref, nmb_ref, decW_ref, decb_ref,
                 out_ref,
                 hist_ref, nmsg_ref, nbr_ref, roeff_ref, aq_ref, cq_ref,
                 deg_ref, sl_ref, bf_ref, bm_ref,
                 cnts_ref, degs_ref, sls_ref, lastlvl_ref,
                 pnode_ref, msrc_ref, cntt_ref, lvl_ref,
                 lcnt_ref, wptr_ref, order_ref, bstart_ref, bcnt_ref):
    f32 = _f32

    lane_e = lax.broadcasted_iota(_i32, (E, N), 1)
    oh0 = (e0_ref[:, :] == lane_e).astype(f32)
    oh1 = (e1_ref[:, :] == lane_e).astype(f32)
    c01 = lax.dot_general(oh0, oh1, (((0,), (0,)), ((), ())),
                          preferred_element_type=f32)
    adjf = ((c01 + c01.T) > 0).astype(f32)

    deg_ref[:, :] = jnp.sum(adjf, axis=0, keepdims=True).astype(_i32)

    ii = lax.broadcasted_iota(_i32, (N, N), 0)
    jj = lax.broadcasted_iota(_i32, (N, N), 1)
    tri = (ii <= jj).astype(f32)
    offs = (lax.dot_general(adjf, tri, (((1,), (0,)), ((), ())),
                            preferred_element_type=f32) - 1.0).astype(_i32)

    nbr_ref[:, 0, :] = jnp.where(adjf > 0, offs, -1)

    sm_row = (st_ref[:, :] != 0)
    smf = sm_row.astype(f32)
    spos = (lax.dot_general(smf, tri, (((1,), (0,)), ((), ())),
                            preferred_element_type=f32) - 1.0).astype(_i32)
    eq2 = jnp.logical_and(spos.T == jj, sm_row.T)
    sl_ref[:, :] = jnp.sum(jnp.where(eq2, ii, 0), axis=0, keepdims=True)
    S = jnp.sum(smf.astype(_i32))

    roeff_ref[:, :] = (roW_ref[0:HID, :] + roW_ref[HID:2 * HID, :] +
                       roW_ref[2 * HID:3 * HID, :] + roW_ref[3 * HID:4 * HID, :])

    inv_sqrt = jnp.float32(1.0 / (HID ** 0.5))
    aq_ref[:, :] = lax.dot_general(kW_ref[:, :], qW_ref[:, :],
                                   (((1,), (1,)), ((), ())),
                                   preferred_element_type=f32) * inv_sqrt
    cq_ref[:, :] = lax.dot_general(qb_ref[:, :], kW_ref[:, :],
                                   (((1,), (1,)), ((), ())),
                                   preferred_element_type=f32) * inv_sqrt

    encoded = jnp.dot(xa_ref[:, :], encW_ref[:, :],
                      preferred_element_type=f32) + encb_ref[:, :]
    hist_ref[:, :, :] = jnp.zeros((N + 1, HP, HID), dtype=f32)
    hist_ref[0:N, 0, :] = encoded

    def _smem_init_n(n, _):
        cnts_ref[n] = 1
        lastlvl_ref[n] = -1
        degs_ref[n] = _lane(deg_ref[:, :], n)
        sls_ref[n] = _lane(sl_ref[:, :], n)
        return 0

    lax.fori_loop(0, N, _smem_init_n, 0)

    def _smem_init_m(i, _):
        lcnt_ref[i] = 0
        return 0

    lax.fori_loop(0, MAXM, _smem_init_m, 0)

    lane_n = lax.broadcasted_iota(_i32, (1, N), 1)
    ri = lax.broadcasted_iota(_i32, (HP, 1), 0)

    def _qinit(i, _):
        pnode_ref[i] = sls_ref[i]
        msrc_ref[i] = -1
        return 0

    lax.fori_loop(0, S, _qinit, 0)

    def a_body(carry):
        h, tail, maxlvl = carry
        node = pnode_ref[h]
        src = msrc_ref[h]

        cnt = cnts_ref[node]
        cnts_ref[node] = cnt + 1
        cntt_ref[h] = cnt

        lvl_src = jnp.where(src < 0, -1, lvl_ref[jnp.maximum(src, 0)])
        mylvl = jnp.maximum(lvl_src, lastlvl_ref[node]) + 1
        lvl_ref[h] = mylvl
        lastlvl_ref[node] = mylvl
        lcnt_ref[mylvl] = lcnt_ref[mylvl] + 1
        maxlvl = jnp.maximum(maxlvl, mylvl)

        dn = degs_ref[node]
        crow = nbr_ref[node]
        kmax = jnp.maximum(jnp.minimum(dn, MAXM - tail), 0)

        def _push(jv, _):
            nb = jnp.sum(jnp.where(crow == jv, lane_n,
                                   jnp.zeros_like(lane_n)))
            pnode_ref[tail + jv] = nb
            msrc_ref[tail + jv] = h
            return 0

        lax.fori_loop(0, kmax, _push, 0)
        return h + 1, tail + dn, maxlvl

    def a_cond(carry):
        h, tail, maxlvl = carry
        return jnp.logical_and(h < tail, h < MAXM)

    T, _, maxlvl = lax.while_loop(
        a_cond, a_body, (jnp.int32(0), S, jnp.int32(-1)))
    nlev = maxlvl + 1

    def _wp_body(l, pos):
        wptr_ref[l] = pos
        return pos + lcnt_ref[l]

    lax.fori_loop(0, nlev, _wp_body, jnp.int32(0))

    def _fill_body(t, _):
        L = lvl_ref[t]
        w = wptr_ref[L]
        order_ref[w] = t
        wptr_ref[L] = w + 1
        return 0

    lax.fori_loop(0, T, _fill_body, 0)

    def b_cond(carry):
        l, done, pos, nb = carry
        return l < nlev

    def b_body(carry):
        l, done, pos, nb = carry
        c = lcnt_ref[l]
        take = jnp.minimum(BW, c - done)
        bstart_ref[nb] = pos
        bcnt_ref[nb] = take
        done2 = done + take
        adv = done2 >= c
        return (jnp.where(adv, l + 1, l), jnp.where(adv, 0, done2),
                pos + take, nb + 1)

    _, _, _, NB = lax.while_loop(
        b_cond, b_body,
        (jnp.int32(0), jnp.int32(0), jnp.int32(0), jnp.int32(0)))

    exr = lax.broadcasted_iota(_i32, (HP * BW, BW), 0)
    exc = lax.broadcasted_iota(_i32, (HP * BW, BW), 1)
    EX = ((exr // HP) == exc).astype(f32)
    rmod = lax.rem(lax.broadcasted_iota(_i32, (HP * BW, 1), 0), HP)

    def p_body(b, _):
        p0 = bstart_ref[b]
        bc = bcnt_ref[b]
        nodes = []
        ts = []
        cnts = []
        for i in range(BW):
            valid = i < bc
            oi = order_ref[jnp.minimum(p0 + i, MAXM - 1)]
            t_i = jnp.where(valid, oi, 0)
            node = jnp.where(valid, pnode_ref[t_i], N)
            src = jnp.where(valid, msrc_ref[t_i], -1)
            cnt = jnp.where(valid, cntt_ref[t_i], 1)
            feats = hist_ref[node]
            bf_ref[i * HP:(i + 1) * HP, :] = feats
            fmr = fm_ref[jnp.minimum(node, N - 1)]
            pmr = nmsg_ref[jnp.maximum(src, 0)]
            bm_ref[i:i + 1, :] = jnp.where(src >= 0, pmr, fmr)
            nodes.append(node)
            ts.append(jnp.where(valid, t_i, MAXM))
            cnts.append(cnt)

        bf = bf_ref[:, :]
        bm = bm_ref[:, :]
        kqT = lax.dot_general(bm, aq_ref[:, :], (((1,), (1,)), ((), ())),
                              preferred_element_type=f32) + cq_ref[:, :]
        kqE = jnp.dot(EX, kqT, preferred_element_type=f32)
        st = jnp.sum(bf * kqE, axis=1, keepdims=True)

        cnt8 = jnp.concatenate(
            [jnp.minimum(c, HL).astype(f32).reshape(1, 1) for c in cnts],
            axis=0)
        cntE = jnp.dot(EX, cnt8, preferred_element_type=f32)
        vmask = rmod.astype(f32) < cntE
        sc = jnp.where(vmask, st, _NEG)
        m8 = jnp.concatenate(
            [jnp.max(sc[i * HP:(i + 1) * HP]).reshape(1, 1) for i in range(BW)],
            axis=0)
        mE = jnp.dot(EX, m8, preferred_element_type=f32)
        p = jnp.where(vmask, jnp.exp(sc - mE), 0.0)
        d8 = lax.dot_general(EX, p, (((0,), (0,)), ((), ())),
                             preferred_element_type=f32)
        dE = jnp.dot(EX, d8, preferred_element_type=f32)
        w = bf * (p / dE)
        V8 = lax.dot_general(EX, w, (((0,), (0,)), ((), ())),
                             preferred_element_type=f32)
        ns8 = jnp.dot(V8, roeff_ref[:, :],
                      preferred_element_type=f32) + rob_ref[:, :]
        nm8 = (jnp.dot(ns8, nmW_ref[0:HID, :], preferred_element_type=f32) +
               jnp.dot(bm, nmW_ref[HID:HID + MSG, :],
                       preferred_element_type=f32) + nmb_ref[:, :])

        for i in range(BW):
            wi = lax.rem(cnts[i], HL)
            blend = jnp.where(ri == wi, ns8[i:i + 1, :],
                              bf[i * HP:(i + 1) * HP, :])
            hist_ref[nodes[i]] = blend
            nmsg_ref[ts[i]] = nm8[i:i + 1, :]
        return 0

    lax.fori_loop(0, NB, p_body, 0)

    def _readout(n, acc):
        li = lax.rem(cnts_ref[n] - 1, HL)
        blk = hist_ref[n]
        return acc + jnp.sum(jnp.where(ri == li, blk, 0.0), axis=0,
                             keepdims=True)

    ffr = lax.fori_loop(0, N, _readout, jnp.zeros((1, HID), dtype=f32))
    lg = jnp.dot(ffr, decW_ref[:, :],
                 preferred_element_type=f32) + decb_ref[:, :]
    mx = jnp.max(lg)
    out_ref[:, :] = lg - mx - jnp.log(jnp.sum(jnp.exp(lg - mx)))


def _run(xa, e0, e1, st, fm, enc_W, enc_b, q_W, q_b, k_W, k_b,
         ro_W, ro_b, nm_W, nm_b, dec_W, dec_b, *, interpret=False):
    return pl.pallas_call(
        _gwac_kernel,
        out_shape=jax.ShapeDtypeStruct((1, OUTF), _f32),
        scratch_shapes=[
            pltpu.VMEM((N + 1, HP, HID), _f32),
            pltpu.VMEM((MAXM + 1, 1, MSG), _f32),
            pltpu.VMEM((N, 1, N), _i32),
            pltpu.VMEM((HID, HID), _f32),
            pltpu.VMEM((HID, MSG), _f32),
            pltpu.VMEM((1, HID), _f32),
            pltpu.VMEM((1, N), _i32),
            pltpu.VMEM((1, N), _i32),
            pltpu.VMEM((HP * BW, HID), _f32),
            pltpu.VMEM((BW, MSG), _f32),
            pltpu.SMEM((N,), _i32),
            pltpu.SMEM((N,), _i32),
            pltpu.SMEM((N,), _i32),
            pltpu.SMEM((N,), _i32),
            pltpu.SMEM((MAXM,), _i32),
            pltpu.SMEM((MAXM,), _i32),
            pltpu.SMEM((MAXM,), _i32),
            pltpu.SMEM((MAXM,), _i32),
            pltpu.SMEM((MAXM,), _i32),
            pltpu.SMEM((MAXM,), _i32),
            pltpu.SMEM((MAXM,), _i32),
            pltpu.SMEM((MAXM,), _i32),
            pltpu.SMEM((MAXM,), _i32),
        ],
        interpret=interpret,
    )(xa, e0, e1, st, fm, enc_W, enc_b, q_W, q_b, k_W, k_b,
      ro_W, ro_b, nm_W, nm_b, dec_W, dec_b)


def kernel(xa, edge_index, starts, first_message, enc_W, enc_b, q_W, q_b,
           k_W, k_b, ro_W, ro_b, nm_W, nm_b, dec_W, dec_b):
    e0 = edge_index[0].reshape(E, 1)
    e1 = edge_index[1].reshape(E, 1)
    st = starts.reshape(1, N).astype(_i32)
    fm = first_message.reshape(N, 1, MSG)
    return _run(xa, e0, e1, st, fm,
                enc_W, enc_b.reshape(1, HID), q_W, q_b.reshape(1, HID),
                k_W, k_b.reshape(1, HID), ro_W, ro_b.reshape(1, HID),
                nm_W, nm_b.reshape(1, MSG), dec_W, dec_b.reshape(1, OUTF))


# speedup vs baseline: 132.5310x; 1.1481x over previous
import jax
import jax.numpy as jnp
from jax import lax
from jax.experimental import pallas as pl
from jax.experimental.pallas import tpu as pltpu

N = 128
HID = 128
MSG = 64
HL = 10
HP = 16
MAXM = 1280
E = 512
OUTF = 16
BW = 32

_i32 = jnp.int32
_f32 = jnp.float32
_NEG = -1e30


def _lane(row, idx):
    ln = lax.broadcasted_iota(_i32, row.shape, 1)
    return jnp.sum(jnp.where(ln == idx, row, jnp.zeros_like(row)))


def _gwac_kernel(xa_ref, e0_ref, e1_ref, st_ref, fm_ref,
                 encW_ref, encb_ref, qW_ref, qb_ref, kW_ref, kb_ref,
                 roW_ref, rob_ref, nmW_ref, nmb_ref, decW_ref, decb_ref,
                 out_ref,
                 hist_ref, nmsg_ref, nbr_ref, roeff_ref, aq_ref, cq_ref,
                 deg_ref, sl_ref, bf_ref, bm_ref,
                 cnts_ref, degs_ref, sls_ref, lastlvl_ref,
                 pnode_ref, msrc_ref, cntt_ref, lvl_ref,
                 lcnt_ref, wptr_ref, order_ref, bstart_ref, bcnt_ref):
    f32 = _f32

    lane_e = lax.broadcasted_iota(_i32, (E, N), 1)
    oh0 = (e0_ref[:, :] == lane_e).astype(f32)
    oh1 = (e1_ref[:, :] == lane_e).astype(f32)
    c01 = lax.dot_general(oh0, oh1, (((0,), (0,)), ((), ())),
                          preferred_element_type=f32)
    adjf = ((c01 + c01.T) > 0).astype(f32)

    deg_ref[:, :] = jnp.sum(adjf, axis=0, keepdims=True).astype(_i32)

    ii = lax.broadcasted_iota(_i32, (N, N), 0)
    jj = lax.broadcasted_iota(_i32, (N, N), 1)
    tri = (ii <= jj).astype(f32)
    offs = (lax.dot_general(adjf, tri, (((1,), (0,)), ((), ())),
                            preferred_element_type=f32) - 1.0).astype(_i32)

    nbr_ref[:, 0, :] = jnp.where(adjf > 0, offs, -1)

    sm_row = (st_ref[:, :] != 0)
    smf = sm_row.astype(f32)
    spos = (lax.dot_general(smf, tri, (((1,), (0,)), ((), ())),
                            preferred_element_type=f32) - 1.0).astype(_i32)
    eq2 = jnp.logical_and(spos.T == jj, sm_row.T)
    sl_ref[:, :] = jnp.sum(jnp.where(eq2, ii, 0), axis=0, keepdims=True)
    S = jnp.sum(smf.astype(_i32))

    roeff_ref[:, :] = (roW_ref[0:HID, :] + roW_ref[HID:2 * HID, :] +
                       roW_ref[2 * HID:3 * HID, :] + roW_ref[3 * HID:4 * HID, :])

    inv_sqrt = jnp.float32(1.0 / (HID ** 0.5))
    aq_ref[:, :] = lax.dot_general(kW_ref[:, :], qW_ref[:, :],
                                   (((1,), (1,)), ((), ())),
                                   preferred_element_type=f32) * inv_sqrt
    cq_ref[:, :] = lax.dot_general(qb_ref[:, :], kW_ref[:, :],
                                   (((1,), (1,)), ((), ())),
                                   preferred_element_type=f32) * inv_sqrt

    encoded = jnp.dot(xa_ref[:, :], encW_ref[:, :],
                      preferred_element_type=f32) + encb_ref[:, :]
    hist_ref[:, :, :] = jnp.zeros((N + 1, HP, HID), dtype=f32)
    hist_ref[0:N, 0, :] = encoded

    def _smem_init_n(n, _):
        cnts_ref[n] = 1
        lastlvl_ref[n] = -1
        degs_ref[n] = _lane(deg_ref[:, :], n)
        sls_ref[n] = _lane(sl_ref[:, :], n)
        return 0

    lax.fori_loop(0, N, _smem_init_n, 0)

    def _smem_init_m(i, _):
        lcnt_ref[i] = 0
        return 0

    lax.fori_loop(0, MAXM, _smem_init_m, 0)

    lane_n = lax.broadcasted_iota(_i32, (1, N), 1)
    ri = lax.broadcasted_iota(_i32, (HP, 1), 0)

    def _qinit(i, _):
        pnode_ref[i] = sls_ref[i]
        msrc_ref[i] = -1
        return 0

    lax.fori_loop(0, S, _qinit, 0)

    def a_body(carry):
        h, tail, maxlvl = carry
        node = pnode_ref[h]
        src = msrc_ref[h]

        cnt = cnts_ref[node]
        cnts_ref[node] = cnt + 1
        cntt_ref[h] = cnt

        lvl_src = jnp.where(src < 0, -1, lvl_ref[jnp.maximum(src, 0)])
        mylvl = jnp.maximum(lvl_src, lastlvl_ref[node]) + 1
        lvl_ref[h] = mylvl
        lastlvl_ref[node] = mylvl
        lcnt_ref[mylvl] = lcnt_ref[mylvl] + 1
        maxlvl = jnp.maximum(maxlvl, mylvl)

        dn = degs_ref[node]
        crow = nbr_ref[node]
        kmax = jnp.maximum(jnp.minimum(dn, MAXM - tail), 0)

        def _push(jv, _):
            nb = jnp.sum(jnp.where(crow == jv, lane_n,
                                   jnp.zeros_like(lane_n)))
            pnode_ref[tail + jv] = nb
            msrc_ref[tail + jv] = h
            return 0

        lax.fori_loop(0, kmax, _push, 0)
        return h + 1, tail + dn, maxlvl

    def a_cond(carry):
        h, tail, maxlvl = carry
        return jnp.logical_and(h < tail, h < MAXM)

    T, _, maxlvl = lax.while_loop(
        a_cond, a_body, (jnp.int32(0), S, jnp.int32(-1)))
    nlev = maxlvl + 1

    def _wp_body(l, pos):
        wptr_ref[l] = pos
        return pos + lcnt_ref[l]

    lax.fori_loop(0, nlev, _wp_body, jnp.int32(0))

    def _fill_body(t, _):
        L = lvl_ref[t]
        w = wptr_ref[L]
        order_ref[w] = t
        wptr_ref[L] = w + 1
        return 0

    lax.fori_loop(0, T, _fill_body, 0)

    def b_cond(carry):
        l, done, pos, nb = carry
        return l < nlev

    def b_body(carry):
        l, done, pos, nb = carry
        c = lcnt_ref[l]
        take = jnp.minimum(BW, c - done)
        bstart_ref[nb] = pos
        bcnt_ref[nb] = take
        done2 = done + take
        adv = done2 >= c
        return (jnp.where(adv, l + 1, l), jnp.where(adv, 0, done2),
                pos + take, nb + 1)

    _, _, _, NB = lax.while_loop(
        b_cond, b_body,
        (jnp.int32(0), jnp.int32(0), jnp.int32(0), jnp.int32(0)))

    exr = lax.broadcasted_iota(_i32, (HP * BW, BW), 0)
    exc = lax.broadcasted_iota(_i32, (HP * BW, BW), 1)
    EX = ((exr // HP) == exc).astype(f32)
    rmod = lax.rem(lax.broadcasted_iota(_i32, (HP * BW, 1), 0), HP)

    def p_body(b, _):
        p0 = bstart_ref[b]
        bc = bcnt_ref[b]
        nodes = []
        ts = []
        cnts = []
        for i in range(BW):
            valid = i < bc
            oi = order_ref[jnp.minimum(p0 + i, MAXM - 1)]
            t_i = jnp.where(valid, oi, 0)
            node = jnp.where(valid, pnode_ref[t_i], N)
            src = jnp.where(valid, msrc_ref[t_i], -1)
            cnt = jnp.where(valid, cntt_ref[t_i], 1)
            feats = hist_ref[node]
            bf_ref[i * HP:(i + 1) * HP, :] = feats
            fmr = fm_ref[jnp.minimum(node, N - 1)]
            pmr = nmsg_ref[jnp.maximum(src, 0)]
            bm_ref[i:i + 1, :] = jnp.where(src >= 0, pmr, fmr)
            nodes.append(node)
            ts.append(jnp.where(valid, t_i, MAXM))
            cnts.append(cnt)

        bf = bf_ref[:, :]
        bm = bm_ref[:, :]
        kqT = lax.dot_general(bm, aq_ref[:, :], (((1,), (1,)), ((), ())),
                              preferred_element_type=f32) + cq_ref[:, :]
        kqE = jnp.dot(EX, kqT, preferred_element_type=f32)
        st = jnp.sum(bf * kqE, axis=1, keepdims=True)

        cnt8 = jnp.concatenate(
            [jnp.minimum(c, HL).astype(f32).reshape(1, 1) for c in cnts],
            axis=0)
        cntE = jnp.dot(EX, cnt8, preferred_element_type=f32)
        vmask = rmod.astype(f32) < cntE
        sc = jnp.where(vmask, st, _NEG)
        m8 = jnp.concatenate(
            [jnp.max(sc[i * HP:(i + 1) * HP]).reshape(1, 1) for i in range(BW)],
            axis=0)
        mE = jnp.dot(EX, m8, preferred_element_type=f32)
        p = jnp.where(vmask, jnp.exp(sc - mE), 0.0)
        d8 = lax.dot_general(EX, p, (((0,), (0,)), ((), ())),
                             preferred_element_type=f32)
        dE = jnp.dot(EX, d8, preferred_element_type=f32)
        w = bf * (p / dE)
        V8 = lax.dot_general(EX, w, (((0,), (0,)), ((), ())),
                             preferred_element_type=f32)
        ns8 = jnp.dot(V8, roeff_ref[:, :],
                      preferred_element_type=f32) + rob_ref[:, :]
        nm8 = (jnp.dot(ns8, nmW_ref[0:HID, :], preferred_element_type=f32) +
               jnp.dot(bm, nmW_ref[HID:HID + MSG, :],
                       preferred_element_type=f32) + nmb_ref[:, :])

        for i in range(BW):
            wi = lax.rem(cnts[i], HL)
            blend = jnp.where(ri == wi, ns8[i:i + 1, :],
                              bf[i * HP:(i + 1) * HP, :])
            hist_ref[nodes[i]] = blend
            nmsg_ref[ts[i]] = nm8[i:i + 1, :]
        return 0

    lax.fori_loop(0, NB, p_body, 0)

    def _readout(n, acc):
        li = lax.rem(cnts_ref[n] - 1, HL)
        blk = hist_ref[n]
        return acc + jnp.sum(jnp.where(ri == li, blk, 0.0), axis=0,
                             keepdims=True)

    ffr = lax.fori_loop(0, N, _readout, jnp.zeros((1, HID), dtype=f32))
    lg = jnp.dot(ffr, decW_ref[:, :],
                 preferred_element_type=f32) + decb_ref[:, :]
    mx = jnp.max(lg)
    out_ref[:, :] = lg - mx - jnp.log(jnp.sum(jnp.exp(lg - mx)))


def _run(xa, e0, e1, st, fm, enc_W, enc_b, q_W, q_b, k_W, k_b,
         ro_W, ro_b, nm_W, nm_b, dec_W, dec_b, *, interpret=False):
    return pl.pallas_call(
        _gwac_kernel,
        out_shape=jax.ShapeDtypeStruct((1, OUTF), _f32),
        scratch_shapes=[
            pltpu.VMEM((N + 1, HP, HID), _f32),
            pltpu.VMEM((MAXM + 1, 1, MSG), _f32),
            pltpu.VMEM((N, 1, N), _i32),
            pltpu.VMEM((HID, HID), _f32),
            pltpu.VMEM((HID, MSG), _f32),
            pltpu.VMEM((1, HID), _f32),
            pltpu.VMEM((1, N), _i32),
            pltpu.VMEM((1, N), _i32),
            pltpu.VMEM((HP * BW, HID), _f32),
            pltpu.VMEM((BW, MSG), _f32),
            pltpu.SMEM((N,), _i32),
            pltpu.SMEM((N,), _i32),
            pltpu.SMEM((N,), _i32),
            pltpu.SMEM((N,), _i32),
            pltpu.SMEM((MAXM,), _i32),
            pltpu.SMEM((MAXM,), _i32),
            pltpu.SMEM((MAXM,), _i32),
            pltpu.SMEM((MAXM,), _i32),
            pltpu.SMEM((MAXM,), _i32),
            pltpu.SMEM((MAXM,), _i32),
            pltpu.SMEM((MAXM,), _i32),
            pltpu.SMEM((MAXM,), _i32),
            pltpu.SMEM((MAXM,), _i32),
        ],
        interpret=interpret,
    )(xa, e0, e1, st, fm, enc_W, enc_b, q_W, q_b, k_W, k_b,
      ro_W, ro_b, nm_W, nm_b, dec_W, dec_b)


def kernel(xa, edge_index, starts, first_message, enc_W, enc_b, q_W, q_b,
           k_W, k_b, ro_W, ro_b, nm_W, nm_b, dec_W, dec_b):
    e0 = edge_index[0].reshape(E, 1)
    e1 = edge_index[1].reshape(E, 1)
    st = starts.reshape(1, N).astype(_i32)
    fm = first_message.reshape(N, 1, MSG)
    return _run(xa, e0, e1, st, fm,
                enc_W, enc_b.reshape(1, HID), q_W, q_b.reshape(1, HID),
                k_W, k_b.reshape(1, HID), ro_W, ro_b.reshape(1, HID),
                nm_W, nm_b.reshape(1, MSG), dec_W, dec_b.reshape(1, OUTF))


# speedup vs baseline: 206.0034x; 1.5544x over previous
import jax
import jax.numpy as jnp
from jax import lax
from jax.experimental import pallas as pl
from jax.experimental.pallas import tpu as pltpu

N = 128
HID = 128
MSG = 64
HL = 10
HP = 16
MAXM = 1280
E = 512
OUTF = 16
BW = 32

_i32 = jnp.int32
_f32 = jnp.float32
_NEG = -1e30


def _lane(row, idx):
    ln = lax.broadcasted_iota(_i32, row.shape, 1)
    return jnp.sum(jnp.where(ln == idx, row, jnp.zeros_like(row)))


def _gwac_kernel(xa_ref, e0_ref, e1_ref, st_ref, fm_ref,
                 encW_ref, encb_ref, qW_ref, qb_ref, kW_ref, kb_ref,
                 roW_ref, rob_ref, nmW_ref, nmb_ref, decW_ref, decb_ref,
                 out_ref,
                 hist_ref, nmsg_ref, nbr_ref, roeff_ref, aq_ref, cq_ref,
                 deg_ref, sl_ref, bf_ref, bm_ref,
                 cnts_ref, degs_ref, sls_ref, lastlvl_ref,
                 pnode_ref, msrc_ref, cntt_ref, lvl_ref,
                 lcnt_ref, wptr_ref, order_ref, bstart_ref, bcnt_ref):
    f32 = _f32

    lane_e = lax.broadcasted_iota(_i32, (E, N), 1)
    oh0 = (e0_ref[:, :] == lane_e).astype(f32)
    oh1 = (e1_ref[:, :] == lane_e).astype(f32)
    c01 = lax.dot_general(oh0, oh1, (((0,), (0,)), ((), ())),
                          preferred_element_type=f32)
    adjf = ((c01 + c01.T) > 0).astype(f32)

    deg_ref[:, :] = jnp.sum(adjf, axis=0, keepdims=True).astype(_i32)

    ii = lax.broadcasted_iota(_i32, (N, N), 0)
    jj = lax.broadcasted_iota(_i32, (N, N), 1)
    tri = (ii <= jj).astype(f32)
    offs = (lax.dot_general(adjf, tri, (((1,), (0,)), ((), ())),
                            preferred_element_type=f32) - 1.0).astype(_i32)

    nbr_ref[:, 0, :] = jnp.where(adjf > 0, offs, -1)

    sm_row = (st_ref[:, :] != 0)
    smf = sm_row.astype(f32)
    spos = (lax.dot_general(smf, tri, (((1,), (0,)), ((), ())),
                            preferred_element_type=f32) - 1.0).astype(_i32)
    eq2 = jnp.logical_and(spos.T == jj, sm_row.T)
    sl_ref[:, :] = jnp.sum(jnp.where(eq2, ii, 0), axis=0, keepdims=True)
    S = jnp.sum(smf.astype(_i32))

    roeff_ref[:, :] = (roW_ref[0:HID, :] + roW_ref[HID:2 * HID, :] +
                       roW_ref[2 * HID:3 * HID, :] + roW_ref[3 * HID:4 * HID, :])

    inv_sqrt = jnp.float32(1.0 / (HID ** 0.5))
    aq_ref[:, :] = lax.dot_general(kW_ref[:, :], qW_ref[:, :],
                                   (((1,), (1,)), ((), ())),
                                   preferred_element_type=f32) * inv_sqrt
    cq_ref[:, :] = lax.dot_general(qb_ref[:, :], kW_ref[:, :],
                                   (((1,), (1,)), ((), ())),
                                   preferred_element_type=f32) * inv_sqrt

    encoded = jnp.dot(xa_ref[:, :], encW_ref[:, :],
                      preferred_element_type=f32) + encb_ref[:, :]
    hist_ref[:, :, :] = jnp.zeros((N + 1, HP, HID), dtype=f32)
    hist_ref[0:N, 0, :] = encoded

    def _smem_init_n(n, _):
        cnts_ref[n] = 1
        lastlvl_ref[n] = -1
        degs_ref[n] = _lane(deg_ref[:, :], n)
        sls_ref[n] = _lane(sl_ref[:, :], n)
        return 0

    lax.fori_loop(0, N, _smem_init_n, 0)

    def _smem_init_m(i, _):
        lcnt_ref[i] = 0
        return 0

    lax.fori_loop(0, MAXM, _smem_init_m, 0)

    lane_n = lax.broadcasted_iota(_i32, (1, N), 1)
    ri = lax.broadcasted_iota(_i32, (HP, 1), 0)

    def _qinit(i, _):
        pnode_ref[i] = sls_ref[i]
        msrc_ref[i] = -1
        return 0

    lax.fori_loop(0, S, _qinit, 0)

    def a_body(carry):
        h, tail, maxlvl = carry
        node = pnode_ref[h]
        src = msrc_ref[h]

        cnt = cnts_ref[node]
        cnts_ref[node] = cnt + 1
        cntt_ref[h] = cnt

        lvl_src = jnp.where(src < 0, -1, lvl_ref[jnp.maximum(src, 0)])
        mylvl = jnp.maximum(lvl_src, lastlvl_ref[node]) + 1
        lvl_ref[h] = mylvl
        lastlvl_ref[node] = mylvl
        lcnt_ref[mylvl] = lcnt_ref[mylvl] + 1
        maxlvl = jnp.maximum(maxlvl, mylvl)

        dn = degs_ref[node]
        crow = nbr_ref[node]
        kmax = jnp.maximum(jnp.minimum(dn, MAXM - tail), 0)

        def _push_chunk(c, _):
            for jo in range(16):
                jv = c * 16 + jo
                nb = jnp.sum(jnp.where(crow == jv, lane_n,
                                       jnp.zeros_like(lane_n)))
                idx = jnp.where(jv < kmax, tail + jv, MAXM + jo)
                pnode_ref[idx] = nb
                msrc_ref[idx] = h
            return 0

        lax.fori_loop(0, (kmax + 15) // 16, _push_chunk, 0)
        return h + 1, tail + dn, maxlvl

    def a_cond(carry):
        h, tail, maxlvl = carry
        return jnp.logical_and(h < tail, h < MAXM)

    T, _, maxlvl = lax.while_loop(
        a_cond, a_body, (jnp.int32(0), S, jnp.int32(-1)))
    nlev = maxlvl + 1

    def _wp_body(l, pos):
        wptr_ref[l] = pos
        return pos + lcnt_ref[l]

    lax.fori_loop(0, nlev, _wp_body, jnp.int32(0))

    def _fill_body(t, _):
        L = lvl_ref[t]
        w = wptr_ref[L]
        order_ref[w] = t
        wptr_ref[L] = w + 1
        return 0

    lax.fori_loop(0, T, _fill_body, 0)

    def b_cond(carry):
        l, done, pos, nb = carry
        return l < nlev

    def b_body(carry):
        l, done, pos, nb = carry
        c = lcnt_ref[l]
        take = jnp.minimum(BW, c - done)
        bstart_ref[nb] = pos
        bcnt_ref[nb] = take
        done2 = done + take
        adv = done2 >= c
        return (jnp.where(adv, l + 1, l), jnp.where(adv, 0, done2),
                pos + take, nb + 1)

    _, _, _, NB = lax.while_loop(
        b_cond, b_body,
        (jnp.int32(0), jnp.int32(0), jnp.int32(0), jnp.int32(0)))

    exr = lax.broadcasted_iota(_i32, (HP * BW, BW), 0)
    exc = lax.broadcasted_iota(_i32, (HP * BW, BW), 1)
    EX = ((exr // HP) == exc).astype(f32)
    rmod = lax.rem(lax.broadcasted_iota(_i32, (HP * BW, 1), 0), HP)

    def p_body(b, _):
        p0 = bstart_ref[b]
        bc = bcnt_ref[b]
        nodes = []
        ts = []
        cnts = []
        for i in range(BW):
            valid = i < bc
            oi = order_ref[jnp.minimum(p0 + i, MAXM - 1)]
            t_i = jnp.where(valid, oi, 0)
            node = jnp.where(valid, pnode_ref[t_i], N)
            src = jnp.where(valid, msrc_ref[t_i], -1)
            cnt = jnp.where(valid, cntt_ref[t_i], 1)
            feats = hist_ref[node]
            bf_ref[i * HP:(i + 1) * HP, :] = feats
            fmr = fm_ref[jnp.minimum(node, N - 1)]
            pmr = nmsg_ref[jnp.maximum(src, 0)]
            bm_ref[i:i + 1, :] = jnp.where(src >= 0, pmr, fmr)
            nodes.append(node)
            ts.append(jnp.where(valid, t_i, MAXM))
            cnts.append(cnt)

        bf = bf_ref[:, :]
        bm = bm_ref[:, :]
        kqT = lax.dot_general(bm, aq_ref[:, :], (((1,), (1,)), ((), ())),
                              preferred_element_type=f32) + cq_ref[:, :]
        kqE = jnp.dot(EX, kqT, preferred_element_type=f32)
        st = jnp.sum(bf * kqE, axis=1, keepdims=True)

        cnt8 = jnp.concatenate(
            [jnp.minimum(c, HL).astype(f32).reshape(1, 1) for c in cnts],
            axis=0)
        cntE = jnp.dot(EX, cnt8, preferred_element_type=f32)
        vmask = rmod.astype(f32) < cntE
        sc = jnp.where(vmask, st, _NEG)
        m8 = jnp.concatenate(
            [jnp.max(sc[i * HP:(i + 1) * HP]).reshape(1, 1) for i in range(BW)],
            axis=0)
        mE = jnp.dot(EX, m8, preferred_element_type=f32)
        p = jnp.where(vmask, jnp.exp(sc - mE), 0.0)
        d8 = lax.dot_general(EX, p, (((0,), (0,)), ((), ())),
                             preferred_element_type=f32)
        dE = jnp.dot(EX, d8, preferred_element_type=f32)
        w = bf * (p / dE)
        V8 = lax.dot_general(EX, w, (((0,), (0,)), ((), ())),
                             preferred_element_type=f32)
        ns8 = jnp.dot(V8, roeff_ref[:, :],
                      preferred_element_type=f32) + rob_ref[:, :]
        nm8 = (jnp.dot(ns8, nmW_ref[0:HID, :], preferred_element_type=f32) +
               jnp.dot(bm, nmW_ref[HID:HID + MSG, :],
                       preferred_element_type=f32) + nmb_ref[:, :])

        for i in range(BW):
            wi = lax.rem(cnts[i], HL)
            blend = jnp.where(ri == wi, ns8[i:i + 1, :],
                              bf[i * HP:(i + 1) * HP, :])
            hist_ref[nodes[i]] = blend
            nmsg_ref[ts[i]] = nm8[i:i + 1, :]
        return 0

    lax.fori_loop(0, NB, p_body, 0)

    def _readout(n, acc):
        li = lax.rem(cnts_ref[n] - 1, HL)
        blk = hist_ref[n]
        return acc + jnp.sum(jnp.where(ri == li, blk, 0.0), axis=0,
                             keepdims=True)

    ffr = lax.fori_loop(0, N, _readout, jnp.zeros((1, HID), dtype=f32))
    lg = jnp.dot(ffr, decW_ref[:, :],
                 preferred_element_type=f32) + decb_ref[:, :]
    mx = jnp.max(lg)
    out_ref[:, :] = lg - mx - jnp.log(jnp.sum(jnp.exp(lg - mx)))


def _run(xa, e0, e1, st, fm, enc_W, enc_b, q_W, q_b, k_W, k_b,
         ro_W, ro_b, nm_W, nm_b, dec_W, dec_b, *, interpret=False):
    return pl.pallas_call(
        _gwac_kernel,
        out_shape=jax.ShapeDtypeStruct((1, OUTF), _f32),
        scratch_shapes=[
            pltpu.VMEM((N + 1, HP, HID), _f32),
            pltpu.VMEM((MAXM + 1, 1, MSG), _f32),
            pltpu.VMEM((N, 1, N), _i32),
            pltpu.VMEM((HID, HID), _f32),
            pltpu.VMEM((HID, MSG), _f32),
            pltpu.VMEM((1, HID), _f32),
            pltpu.VMEM((1, N), _i32),
            pltpu.VMEM((1, N), _i32),
            pltpu.VMEM((HP * BW, HID), _f32),
            pltpu.VMEM((BW, MSG), _f32),
            pltpu.SMEM((N,), _i32),
            pltpu.SMEM((N,), _i32),
            pltpu.SMEM((N,), _i32),
            pltpu.SMEM((N,), _i32),
            pltpu.SMEM((MAXM + 16,), _i32),
            pltpu.SMEM((MAXM + 16,), _i32),
            pltpu.SMEM((MAXM,), _i32),
            pltpu.SMEM((MAXM,), _i32),
            pltpu.SMEM((MAXM,), _i32),
            pltpu.SMEM((MAXM,), _i32),
            pltpu.SMEM((MAXM,), _i32),
            pltpu.SMEM((MAXM,), _i32),
            pltpu.SMEM((MAXM,), _i32),
        ],
        interpret=interpret,
    )(xa, e0, e1, st, fm, enc_W, enc_b, q_W, q_b, k_W, k_b,
      ro_W, ro_b, nm_W, nm_b, dec_W, dec_b)


def kernel(xa, edge_index, starts, first_message, enc_W, enc_b, q_W, q_b,
           k_W, k_b, ro_W, ro_b, nm_W, nm_b, dec_W, dec_b):
    e0 = edge_index[0].reshape(E, 1)
    e1 = edge_index[1].reshape(E, 1)
    st = starts.reshape(1, N).astype(_i32)
    fm = first_message.reshape(N, 1, MSG)
    return _run(xa, e0, e1, st, fm,
                enc_W, enc_b.reshape(1, HID), q_W, q_b.reshape(1, HID),
                k_W, k_b.reshape(1, HID), ro_W, ro_b.reshape(1, HID),
                nm_W, nm_b.reshape(1, MSG), dec_W, dec_b.reshape(1, OUTF))


# speedup vs baseline: 223.0702x; 1.0828x over previous
import jax
import jax.numpy as jnp
from jax import lax
from jax.experimental import pallas as pl
from jax.experimental.pallas import tpu as pltpu

N = 128
HID = 128
MSG = 64
HL = 10
HP = 16
MAXM = 1280
E = 512
OUTF = 16
BW = 32

_i32 = jnp.int32
_f32 = jnp.float32
_NEG = -1e30


def _lane(row, idx):
    ln = lax.broadcasted_iota(_i32, row.shape, 1)
    return jnp.sum(jnp.where(ln == idx, row, jnp.zeros_like(row)))


def _gwac_kernel(xa_ref, e0_ref, e1_ref, st_ref, fm_ref,
                 encW_ref, encb_ref, qW_ref, qb_ref, kW_ref, kb_ref,
                 roW_ref, rob_ref, nmW_ref, nmb_ref, decW_ref, decb_ref,
                 out_ref,
                 hist_ref, nmsg_ref, nbr_ref, roeff_ref, aq_ref, cq_ref,
                 deg_ref, sl_ref, bf_ref, bm_ref,
                 cnts_ref, degs_ref, sls_ref, lastlvl_ref,
                 pnode_ref, msrc_ref, cntt_ref, lvl_ref,
                 lcnt_ref, wptr_ref, order_ref, bstart_ref, bcnt_ref):
    f32 = _f32

    lane_e = lax.broadcasted_iota(_i32, (E, N), 1)
    oh0 = (e0_ref[:, :] == lane_e).astype(f32)
    oh1 = (e1_ref[:, :] == lane_e).astype(f32)
    c01 = lax.dot_general(oh0, oh1, (((0,), (0,)), ((), ())),
                          preferred_element_type=f32)
    adjf = ((c01 + c01.T) > 0).astype(f32)

    deg_ref[:, :] = jnp.sum(adjf, axis=0, keepdims=True).astype(_i32)

    ii = lax.broadcasted_iota(_i32, (N, N), 0)
    jj = lax.broadcasted_iota(_i32, (N, N), 1)
    tri = (ii <= jj).astype(f32)
    offs = (lax.dot_general(adjf, tri, (((1,), (0,)), ((), ())),
                            preferred_element_type=f32) - 1.0).astype(_i32)

    nbr_ref[:, 0, :] = jnp.where(adjf > 0, offs, -1)

    sm_row = (st_ref[:, :] != 0)
    smf = sm_row.astype(f32)
    spos = (lax.dot_general(smf, tri, (((1,), (0,)), ((), ())),
                            preferred_element_type=f32) - 1.0).astype(_i32)
    eq2 = jnp.logical_and(spos.T == jj, sm_row.T)
    sl_ref[:, :] = jnp.sum(jnp.where(eq2, ii, 0), axis=0, keepdims=True)
    S = jnp.sum(smf.astype(_i32))

    roeff_ref[:, :] = (roW_ref[0:HID, :] + roW_ref[HID:2 * HID, :] +
                       roW_ref[2 * HID:3 * HID, :] + roW_ref[3 * HID:4 * HID, :])

    inv_sqrt = jnp.float32(1.0 / (HID ** 0.5))
    aq_ref[:, :] = lax.dot_general(kW_ref[:, :], qW_ref[:, :],
                                   (((1,), (1,)), ((), ())),
                                   preferred_element_type=f32) * inv_sqrt
    cq_ref[:, :] = lax.dot_general(qb_ref[:, :], kW_ref[:, :],
                                   (((1,), (1,)), ((), ())),
                                   preferred_element_type=f32) * inv_sqrt

    encoded = jnp.dot(xa_ref[:, :], encW_ref[:, :],
                      preferred_element_type=f32) + encb_ref[:, :]
    hist_ref[:, :, :] = jnp.zeros((N + 1, HP, HID), dtype=f32)
    hist_ref[0:N, 0, :] = encoded

    def _smem_init_n(n, _):
        cnts_ref[n] = 1
        lastlvl_ref[n] = -1
        degs_ref[n] = _lane(deg_ref[:, :], n)
        sls_ref[n] = _lane(sl_ref[:, :], n)
        return 0

    lax.fori_loop(0, N, _smem_init_n, 0)

    def _smem_init_m(i, _):
        lcnt_ref[i] = 0
        return 0

    lax.fori_loop(0, MAXM, _smem_init_m, 0)

    lane_n = lax.broadcasted_iota(_i32, (1, N), 1)
    ri = lax.broadcasted_iota(_i32, (HP, 1), 0)

    def _qinit(i, _):
        pnode_ref[i] = sls_ref[i]
        msrc_ref[i] = -1
        return 0

    lax.fori_loop(0, S, _qinit, 0)

    def a_body(carry):
        h, tail, maxlvl = carry
        node = pnode_ref[h]
        src = msrc_ref[h]

        cnt = cnts_ref[node]
        cnts_ref[node] = cnt + 1
        cntt_ref[h] = cnt

        lvl_src = jnp.where(src < 0, -1, lvl_ref[jnp.maximum(src, 0)])
        mylvl = jnp.maximum(lvl_src, lastlvl_ref[node]) + 1
        lvl_ref[h] = mylvl
        lastlvl_ref[node] = mylvl
        lcnt_ref[mylvl] = lcnt_ref[mylvl] + 1
        maxlvl = jnp.maximum(maxlvl, mylvl)

        dn = degs_ref[node]
        crow = nbr_ref[node]
        kmax = jnp.maximum(jnp.minimum(dn, MAXM - tail), 0)

        def _push_chunk(c, _):
            for jo in range(16):
                jv = c * 16 + jo
                nb = jnp.sum(jnp.where(crow == jv, lane_n,
                                       jnp.zeros_like(lane_n)))
                idx = jnp.where(jv < kmax, tail + jv, MAXM + jo)
                pnode_ref[idx] = nb
                msrc_ref[idx] = h
            return 0

        lax.fori_loop(0, (kmax + 15) // 16, _push_chunk, 0)
        return h + 1, tail + dn, maxlvl

    def a_cond(carry):
        h, tail, maxlvl = carry
        return jnp.logical_and(h < tail, h < MAXM)

    T, _, maxlvl = lax.while_loop(
        a_cond, a_body, (jnp.int32(0), S, jnp.int32(-1)))
    nlev = maxlvl + 1

    def _wp_body(l, pos):
        wptr_ref[l] = pos
        return pos + lcnt_ref[l]

    lax.fori_loop(0, nlev, _wp_body, jnp.int32(0))

    def _fill_body(t, _):
        L = lvl_ref[t]
        w = wptr_ref[L]
        order_ref[w] = t
        wptr_ref[L] = w + 1
        return 0

    lax.fori_loop(0, T, _fill_body, 0)

    def b_cond(carry):
        l, done, pos, nb = carry
        return l < nlev

    def b_body(carry):
        l, done, pos, nb = carry
        c = lcnt_ref[l]
        take = jnp.minimum(BW, c - done)
        bstart_ref[nb] = pos
        bcnt_ref[nb] = take
        done2 = done + take
        adv = done2 >= c
        return (jnp.where(adv, l + 1, l), jnp.where(adv, 0, done2),
                pos + take, nb + 1)

    _, _, _, NB = lax.while_loop(
        b_cond, b_body,
        (jnp.int32(0), jnp.int32(0), jnp.int32(0), jnp.int32(0)))

    exr = lax.broadcasted_iota(_i32, (HP * BW, BW), 0)
    exc = lax.broadcasted_iota(_i32, (HP * BW, BW), 1)
    EX = ((exr // HP) == exc).astype(f32)
    rmod = lax.rem(lax.broadcasted_iota(_i32, (HP * BW, 1), 0), HP)

    def p_body(b, _):
        p0 = bstart_ref[b]
        bc = bcnt_ref[b]
        nodes = []
        ts = []
        cnts = []
        for i in range(BW):
            valid = i < bc
            oi = order_ref[jnp.minimum(p0 + i, MAXM - 1)]
            t_i = jnp.where(valid, oi, 0)
            node = jnp.where(valid, pnode_ref[t_i], N)
            src = jnp.where(valid, msrc_ref[t_i], -1)
            cnt = jnp.where(valid, cntt_ref[t_i], 1)
            feats = hist_ref[node]
            bf_ref[i * HP:(i + 1) * HP, :] = feats
            fmr = fm_ref[jnp.minimum(node, N - 1)]
            pmr = nmsg_ref[jnp.maximum(src, 0)]
            bm_ref[i:i + 1, :] = jnp.where(src >= 0, pmr, fmr)
            nodes.append(node)
            ts.append(jnp.where(valid, t_i, MAXM))
            cnts.append(cnt)

        bf = bf_ref[:, :]
        bm = bm_ref[:, :]
        kqT = lax.dot_general(bm, aq_ref[:, :], (((1,), (1,)), ((), ())),
                              preferred_element_type=f32) + cq_ref[:, :]
        kqE = jnp.dot(EX, kqT, preferred_element_type=f32)
        st = jnp.sum(bf * kqE, axis=1, keepdims=True)

        cnt8 = jnp.concatenate(
            [jnp.minimum(c, HL).astype(f32).reshape(1, 1) for c in cnts],
            axis=0)
        cntE = jnp.dot(EX, cnt8, preferred_element_type=f32)
        vmask = rmod.astype(f32) < cntE
        sc = jnp.where(vmask, st, _NEG)
        m8 = jnp.concatenate(
            [jnp.max(sc[i * HP:(i + 1) * HP]).reshape(1, 1) for i in range(BW)],
            axis=0)
        mE = jnp.dot(EX, m8, preferred_element_type=f32)
        p = jnp.where(vmask, jnp.exp(sc - mE), 0.0)
        d8 = lax.dot_general(EX, p, (((0,), (0,)), ((), ())),
                             preferred_element_type=f32)
        w = bf * p
        V8 = lax.dot_general(EX, w, (((0,), (0,)), ((), ())),
                             preferred_element_type=f32) / d8
        ns8 = jnp.dot(V8, roeff_ref[:, :],
                      preferred_element_type=f32) + rob_ref[:, :]
        nm8 = (jnp.dot(ns8, nmW_ref[0:HID, :], preferred_element_type=f32) +
               jnp.dot(bm, nmW_ref[HID:HID + MSG, :],
                       preferred_element_type=f32) + nmb_ref[:, :])

        for i in range(BW):
            wi = lax.rem(cnts[i], HL)
            hist_ref[nodes[i], pl.ds(wi, 1), :] = ns8[i:i + 1, :]
            nmsg_ref[ts[i]] = nm8[i:i + 1, :]
        return 0

    lax.fori_loop(0, NB, p_body, 0)

    def _readout(n, acc):
        li = lax.rem(cnts_ref[n] - 1, HL)
        blk = hist_ref[n]
        return acc + jnp.sum(jnp.where(ri == li, blk, 0.0), axis=0,
                             keepdims=True)

    ffr = lax.fori_loop(0, N, _readout, jnp.zeros((1, HID), dtype=f32))
    lg = jnp.dot(ffr, decW_ref[:, :],
                 preferred_element_type=f32) + decb_ref[:, :]
    mx = jnp.max(lg)
    out_ref[:, :] = lg - mx - jnp.log(jnp.sum(jnp.exp(lg - mx)))


def _run(xa, e0, e1, st, fm, enc_W, enc_b, q_W, q_b, k_W, k_b,
         ro_W, ro_b, nm_W, nm_b, dec_W, dec_b, *, interpret=False):
    return pl.pallas_call(
        _gwac_kernel,
        out_shape=jax.ShapeDtypeStruct((1, OUTF), _f32),
        scratch_shapes=[
            pltpu.VMEM((N + 1, HP, HID), _f32),
            pltpu.VMEM((MAXM + 1, 1, MSG), _f32),
            pltpu.VMEM((N, 1, N), _i32),
            pltpu.VMEM((HID, HID), _f32),
            pltpu.VMEM((HID, MSG), _f32),
            pltpu.VMEM((1, HID), _f32),
            pltpu.VMEM((1, N), _i32),
            pltpu.VMEM((1, N), _i32),
            pltpu.VMEM((HP * BW, HID), _f32),
            pltpu.VMEM((BW, MSG), _f32),
            pltpu.SMEM((N,), _i32),
            pltpu.SMEM((N,), _i32),
            pltpu.SMEM((N,), _i32),
            pltpu.SMEM((N,), _i32),
            pltpu.SMEM((MAXM + 16,), _i32),
            pltpu.SMEM((MAXM + 16,), _i32),
            pltpu.SMEM((MAXM,), _i32),
            pltpu.SMEM((MAXM,), _i32),
            pltpu.SMEM((MAXM,), _i32),
            pltpu.SMEM((MAXM,), _i32),
            pltpu.SMEM((MAXM,), _i32),
            pltpu.SMEM((MAXM,), _i32),
            pltpu.SMEM((MAXM,), _i32),
        ],
        interpret=interpret,
    )(xa, e0, e1, st, fm, enc_W, enc_b, q_W, q_b, k_W, k_b,
      ro_W, ro_b, nm_W, nm_b, dec_W, dec_b)


def kernel(xa, edge_index, starts, first_message, enc_W, enc_b, q_W, q_b,
           k_W, k_b, ro_W, ro_b, nm_W, nm_b, dec_W, dec_b):
    e0 = edge_index[0].reshape(E, 1)
    e1 = edge_index[1].reshape(E, 1)
    st = starts.reshape(1, N).astype(_i32)
    fm = first_message.reshape(N, 1, MSG)
    return _run(xa, e0, e1, st, fm,
                enc_W, enc_b.reshape(1, HID), q_W, q_b.reshape(1, HID),
                k_W, k_b.reshape(1, HID), ro_W, ro_b.reshape(1, HID),
                nm_W, nm_b.reshape(1, MSG), dec_W, dec_b.reshape(1, OUTF))


# speedup vs baseline: 225.2990x; 1.0100x over previous
import jax
import jax.numpy as jnp
from jax import lax
from jax.experimental import pallas as pl
from jax.experimental.pallas import tpu as pltpu

N = 128
HID = 128
MSG = 64
HL = 10
HP = 16
MAXM = 1280
E = 512
OUTF = 16
BW = 32

_i32 = jnp.int32
_f32 = jnp.float32
_NEG = -1e30


def _lane(row, idx):
    ln = lax.broadcasted_iota(_i32, row.shape, 1)
    return jnp.sum(jnp.where(ln == idx, row, jnp.zeros_like(row)))


def _gwac_kernel(xa_ref, e0_ref, e1_ref, st_ref, fm_ref,
                 encW_ref, encb_ref, qW_ref, qb_ref, kW_ref, kb_ref,
                 roW_ref, rob_ref, nmW_ref, nmb_ref, decW_ref, decb_ref,
                 out_ref,
                 hist_ref, nmsg_ref, nbr_ref, roeff_ref, aq_ref, cq_ref,
                 deg_ref, sl_ref, bf0_ref, bm0_ref, bf1_ref, bm1_ref,
                 bf2_ref, bm2_ref, bf3_ref, bm3_ref,
                 cnts_ref, degs_ref, sls_ref, lastlvl_ref,
                 pnode_ref, msrc_ref, cntt_ref, lvl_ref,
                 lcnt_ref, wptr_ref, order_ref):
    f32 = _f32

    lane_e = lax.broadcasted_iota(_i32, (E, N), 1)
    oh0 = (e0_ref[:, :] == lane_e).astype(f32)
    oh1 = (e1_ref[:, :] == lane_e).astype(f32)
    c01 = lax.dot_general(oh0, oh1, (((0,), (0,)), ((), ())),
                          preferred_element_type=f32)
    adjf = ((c01 + c01.T) > 0).astype(f32)

    deg_ref[:, :] = jnp.sum(adjf, axis=0, keepdims=True).astype(_i32)

    ii = lax.broadcasted_iota(_i32, (N, N), 0)
    jj = lax.broadcasted_iota(_i32, (N, N), 1)
    tri = (ii <= jj).astype(f32)
    offs = (lax.dot_general(adjf, tri, (((1,), (0,)), ((), ())),
                            preferred_element_type=f32) - 1.0).astype(_i32)

    nbr_ref[:, 0, :] = jnp.where(adjf > 0, offs, -1)

    sm_row = (st_ref[:, :] != 0)
    smf = sm_row.astype(f32)
    spos = (lax.dot_general(smf, tri, (((1,), (0,)), ((), ())),
                            preferred_element_type=f32) - 1.0).astype(_i32)
    eq2 = jnp.logical_and(spos.T == jj, sm_row.T)
    sl_ref[:, :] = jnp.sum(jnp.where(eq2, ii, 0), axis=0, keepdims=True)
    S = jnp.sum(smf.astype(_i32))

    roeff_ref[:, :] = (roW_ref[0:HID, :] + roW_ref[HID:2 * HID, :] +
                       roW_ref[2 * HID:3 * HID, :] + roW_ref[3 * HID:4 * HID, :])

    inv_sqrt = jnp.float32(1.0 / (HID ** 0.5))
    aq_ref[:, :] = lax.dot_general(kW_ref[:, :], qW_ref[:, :],
                                   (((1,), (1,)), ((), ())),
                                   preferred_element_type=f32) * inv_sqrt
    cq_ref[:, :] = lax.dot_general(qb_ref[:, :], kW_ref[:, :],
                                   (((1,), (1,)), ((), ())),
                                   preferred_element_type=f32) * inv_sqrt

    encoded = jnp.dot(xa_ref[:, :], encW_ref[:, :],
                      preferred_element_type=f32) + encb_ref[:, :]
    hist_ref[:, :, :] = jnp.zeros((N + 1, HP, HID), dtype=f32)
    hist_ref[0:N, 0, :] = encoded

    def _smem_init_n(n, _):
        cnts_ref[n] = 1
        lastlvl_ref[n] = -1
        degs_ref[n] = _lane(deg_ref[:, :], n)
        sls_ref[n] = _lane(sl_ref[:, :], n)
        return 0

    lax.fori_loop(0, N, _smem_init_n, 0)

    def _smem_init_m(i, _):
        lcnt_ref[i] = 0
        return 0

    lax.fori_loop(0, MAXM, _smem_init_m, 0)

    lane_n = lax.broadcasted_iota(_i32, (1, N), 1)
    ri = lax.broadcasted_iota(_i32, (HP, 1), 0)

    def _qinit(i, _):
        pnode_ref[i] = sls_ref[i]
        msrc_ref[i] = -1
        return 0

    lax.fori_loop(0, S, _qinit, 0)

    def a_body(carry):
        h, tail, maxlvl = carry
        node = pnode_ref[h]
        src = msrc_ref[h]

        cnt = cnts_ref[node]
        cnts_ref[node] = cnt + 1
        cntt_ref[h] = cnt

        lvl_src = jnp.where(src < 0, -1, lvl_ref[jnp.maximum(src, 0)])
        mylvl = jnp.maximum(lvl_src, lastlvl_ref[node]) + 1
        lvl_ref[h] = mylvl
        lastlvl_ref[node] = mylvl
        lcnt_ref[mylvl] = lcnt_ref[mylvl] + 1
        maxlvl = jnp.maximum(maxlvl, mylvl)

        dn = degs_ref[node]
        crow = nbr_ref[node]
        kmax = jnp.maximum(jnp.minimum(dn, MAXM - tail), 0)

        def _push_chunk(c, _):
            for jo in range(16):
                jv = c * 16 + jo
                nb = jnp.sum(jnp.where(crow == jv, lane_n,
                                       jnp.zeros_like(lane_n)))
                idx = jnp.where(jv < kmax, tail + jv, MAXM + jo)
                pnode_ref[idx] = nb
                msrc_ref[idx] = h
            return 0

        lax.fori_loop(0, (kmax + 15) // 16, _push_chunk, 0)
        return h + 1, tail + dn, maxlvl

    def a_cond(carry):
        h, tail, maxlvl = carry
        return jnp.logical_and(h < tail, h < MAXM)

    T, _, maxlvl = lax.while_loop(
        a_cond, a_body, (jnp.int32(0), S, jnp.int32(-1)))
    nlev = maxlvl + 1

    def _wp_body(l, pos):
        wptr_ref[l] = pos
        return pos + lcnt_ref[l]

    lax.fori_loop(0, nlev, _wp_body, jnp.int32(0))

    def _fill_body(t, _):
        L = lvl_ref[t]
        w = wptr_ref[L]
        order_ref[w] = t
        wptr_ref[L] = w + 1
        return 0

    lax.fori_loop(0, T, _fill_body, 0)

    exr = lax.broadcasted_iota(_i32, (HP * BW, BW), 0)
    exc = lax.broadcasted_iota(_i32, (HP * BW, BW), 1)
    EX = ((exr // HP) == exc).astype(f32)
    rmod = lax.rem(lax.broadcasted_iota(_i32, (HP * BW, 1), 0), HP)

    def _chunk(p0, bc, bf_ref, bm_ref):
        nodes = []
        ts = []
        cnts = []
        for i in range(BW):
            valid = i < bc
            oi = order_ref[jnp.minimum(p0 + i, MAXM - 1)]
            t_i = jnp.where(valid, oi, 0)
            node = jnp.where(valid, pnode_ref[t_i], N)
            src = jnp.where(valid, msrc_ref[t_i], -1)
            cnt = jnp.where(valid, cntt_ref[t_i], 1)
            feats = hist_ref[node]
            bf_ref[i * HP:(i + 1) * HP, :] = feats
            fmr = fm_ref[jnp.minimum(node, N - 1)]
            pmr = nmsg_ref[jnp.maximum(src, 0)]
            bm_ref[i:i + 1, :] = jnp.where(src >= 0, pmr, fmr)
            nodes.append(node)
            ts.append(jnp.where(valid, t_i, MAXM))
            cnts.append(cnt)

        bf = bf_ref[:, :]
        bm = bm_ref[:, :]
        kqT = lax.dot_general(bm, aq_ref[:, :], (((1,), (1,)), ((), ())),
                              preferred_element_type=f32) + cq_ref[:, :]
        kqE = jnp.dot(EX, kqT, preferred_element_type=f32)
        st = jnp.sum(bf * kqE, axis=1, keepdims=True)

        cnt8 = jnp.concatenate(
            [jnp.minimum(c, HL).astype(f32).reshape(1, 1) for c in cnts],
            axis=0)
        cntE = jnp.dot(EX, cnt8, preferred_element_type=f32)
        vmask = rmod.astype(f32) < cntE
        sc = jnp.where(vmask, st, _NEG)
        m8 = jnp.concatenate(
            [jnp.max(sc[i * HP:(i + 1) * HP]).reshape(1, 1) for i in range(BW)],
            axis=0)
        mE = jnp.dot(EX, m8, preferred_element_type=f32)
        p = jnp.where(vmask, jnp.exp(sc - mE), 0.0)
        d8 = lax.dot_general(EX, p, (((0,), (0,)), ((), ())),
                             preferred_element_type=f32)
        w = bf * p
        V8 = lax.dot_general(EX, w, (((0,), (0,)), ((), ())),
                             preferred_element_type=f32) / d8
        ns8 = jnp.dot(V8, roeff_ref[:, :],
                      preferred_element_type=f32) + rob_ref[:, :]
        nm8 = (jnp.dot(ns8, nmW_ref[0:HID, :], preferred_element_type=f32) +
               jnp.dot(bm, nmW_ref[HID:HID + MSG, :],
                       preferred_element_type=f32) + nmb_ref[:, :])

        for i in range(BW):
            wi = lax.rem(cnts[i], HL)
            hist_ref[nodes[i], pl.ds(wi, 1), :] = ns8[i:i + 1, :]
            nmsg_ref[ts[i]] = nm8[i:i + 1, :]

    cbufs = [(bf0_ref, bm0_ref), (bf1_ref, bm1_ref),
             (bf2_ref, bm2_ref), (bf3_ref, bm3_ref)]

    def lvl_body(l, _):
        cnt_l = lcnt_ref[l]
        base = wptr_ref[l] - cnt_l

        def _mk(nchunks):
            def _branch():
                for c in range(nchunks):
                    bfr, bmr = cbufs[c]
                    _chunk(base + c * BW, cnt_l - c * BW, bfr, bmr)
            return _branch

        lax.switch(jnp.minimum((cnt_l - 1) // BW, 3),
                   [_mk(1), _mk(2), _mk(3), _mk(4)])
        return 0

    lax.fori_loop(0, nlev, lvl_body, 0)

    def _readout(n, acc):
        li = lax.rem(cnts_ref[n] - 1, HL)
        blk = hist_ref[n]
        return acc + jnp.sum(jnp.where(ri == li, blk, 0.0), axis=0,
                             keepdims=True)

    ffr = lax.fori_loop(0, N, _readout, jnp.zeros((1, HID), dtype=f32))
    lg = jnp.dot(ffr, decW_ref[:, :],
                 preferred_element_type=f32) + decb_ref[:, :]
    mx = jnp.max(lg)
    out_ref[:, :] = lg - mx - jnp.log(jnp.sum(jnp.exp(lg - mx)))


def _run(xa, e0, e1, st, fm, enc_W, enc_b, q_W, q_b, k_W, k_b,
         ro_W, ro_b, nm_W, nm_b, dec_W, dec_b, *, interpret=False):
    return pl.pallas_call(
        _gwac_kernel,
        out_shape=jax.ShapeDtypeStruct((1, OUTF), _f32),
        scratch_shapes=[
            pltpu.VMEM((N + 1, HP, HID), _f32),
            pltpu.VMEM((MAXM + 1, 1, MSG), _f32),
            pltpu.VMEM((N, 1, N), _i32),
            pltpu.VMEM((HID, HID), _f32),
            pltpu.VMEM((HID, MSG), _f32),
            pltpu.VMEM((1, HID), _f32),
            pltpu.VMEM((1, N), _i32),
            pltpu.VMEM((1, N), _i32),
            pltpu.VMEM((HP * BW, HID), _f32),
            pltpu.VMEM((BW, MSG), _f32),
            pltpu.VMEM((HP * BW, HID), _f32),
            pltpu.VMEM((BW, MSG), _f32),
            pltpu.VMEM((HP * BW, HID), _f32),
            pltpu.VMEM((BW, MSG), _f32),
            pltpu.VMEM((HP * BW, HID), _f32),
            pltpu.VMEM((BW, MSG), _f32),
            pltpu.SMEM((N,), _i32),
            pltpu.SMEM((N,), _i32),
            pltpu.SMEM((N,), _i32),
            pltpu.SMEM((N,), _i32),
            pltpu.SMEM((MAXM + 16,), _i32),
            pltpu.SMEM((MAXM + 16,), _i32),
            pltpu.SMEM((MAXM,), _i32),
            pltpu.SMEM((MAXM,), _i32),
            pltpu.SMEM((MAXM,), _i32),
            pltpu.SMEM((MAXM,), _i32),
            pltpu.SMEM((MAXM,), _i32),
        ],
        interpret=interpret,
    )(xa, e0, e1, st, fm, enc_W, enc_b, q_W, q_b, k_W, k_b,
      ro_W, ro_b, nm_W, nm_b, dec_W, dec_b)


def kernel(xa, edge_index, starts, first_message, enc_W, enc_b, q_W, q_b,
           k_W, k_b, ro_W, ro_b, nm_W, nm_b, dec_W, dec_b):
    e0 = edge_index[0].reshape(E, 1)
    e1 = edge_index[1].reshape(E, 1)
    st = starts.reshape(1, N).astype(_i32)
    fm = first_message.reshape(N, 1, MSG)
    return _run(xa, e0, e1, st, fm,
                enc_W, enc_b.reshape(1, HID), q_W, q_b.reshape(1, HID),
                k_W, k_b.reshape(1, HID), ro_W, ro_b.reshape(1, HID),
                nm_W, nm_b.reshape(1, MSG), dec_W, dec_b.reshape(1, OUTF))
